# Initial kernel scaffold; baseline (speedup 1.0000x reference)
import jax
import jax.numpy as jnp
from jax.experimental import pallas as pl


def kernel(ent_embeds, rel_embeds, W_r1, W_sl1, W_el1, W_r2, W_sl2, W_el2, lin_w, lin_b, gru_w_ih, gru_w_hh, gru_b_ih, gru_b_hh, convR_w, convR_b, fcR_w, fcR_b, convE_w, convE_b, fcE_w, fcE_b, edge_index, edge_rel, subj, rel, obj):
    raise NotImplementedError("write your pallas kernel here")


# speedup vs baseline: 2.2713x; 2.2713x over previous
import functools

import jax
import jax.numpy as jnp
from jax import lax
from jax.experimental import pallas as pl
from jax.experimental.pallas import tpu as pltpu
from jax.experimental.pallas import tpu_sc as plsc

N_ENT = 10000
N_REL = 200
H = 128
CH = 50
K = 3
B = 1024
E = 320000
RRELU_SLOPE = (1.0 / 8.0 + 1.0 / 3.0) / 2.0

NC = 2
NS = 16
NPE = 10240
NPR = 256
RPE = NPE // NS
RPR = NPR // NS
NW = NC * NS
CHUNK = 80
NCHUNK = 128
SUP = 8
NSB = NCHUNK // SUP
EPT = NCHUNK * CHUNK
E_PAD = NW * EPT
BPT = B // NW

_MESH = dict(core_axis_name="c", subcore_axis_name="s", num_cores=NC,
             num_subcores=NS)


def _l2n(x):
    n = jnp.sqrt(jnp.sum(x * x, axis=-1, keepdims=True))
    return x / jnp.maximum(n, 1e-12)


def _lrelu(x):
    return jnp.where(x >= 0, x, RRELU_SLOPE * x)


def _edge_stats(src_r, dst_r, rel_r, ent_e, zsum, zcnt, zdeg, ones16):
    mesh = plsc.VectorSubcoreMesh(**_MESH)

    @functools.partial(
        pl.kernel,
        out_type=[
            jax.ShapeDtypeStruct((NC, NPR, H), jnp.float32),
            jax.ShapeDtypeStruct((NC, NPR, H), jnp.float32),
            jax.ShapeDtypeStruct((NC, NPE, H), jnp.float32),
        ],
        mesh=mesh,
        scratch_types=[
            pltpu.VMEM((SUP, CHUNK), jnp.int32),
            pltpu.VMEM((SUP, CHUNK), jnp.int32),
            pltpu.VMEM((SUP, CHUNK), jnp.int32),
            pltpu.VMEM((CHUNK, H), jnp.float32),
            pltpu.VMEM((CHUNK, H), jnp.float32),
            pltpu.VMEM_SHARED((NPR, H), jnp.float32),
            pltpu.VMEM_SHARED((NPR, H), jnp.float32),
            pltpu.VMEM_SHARED((NPE, H), jnp.float32),
            pltpu.SemaphoreType.DMA,
        ],
    )
    def k(src_h, dst_h, rel_h, ent_h, zs_h, zc_h, zd_h, ones_h,
          sums_o, cnt_o, deg_o,
          src_v, dst_v, rel_v, rows_v, ones_v, sums_s, cnt_s, deg_s, sem):
        cid = lax.axis_index("c")
        sid = lax.axis_index("s")
        wid = cid * NS + sid
        pltpu.sync_copy(ones_h, ones_v)
        pltpu.sync_copy(zd_h.at[pl.ds(sid * RPE, RPE)],
                        deg_s.at[pl.ds(sid * RPE, RPE)])
        pltpu.sync_copy(zs_h.at[pl.ds(sid * RPR, RPR)],
                        sums_s.at[pl.ds(sid * RPR, RPR)])
        pltpu.sync_copy(zc_h.at[pl.ds(sid * RPR, RPR)],
                        cnt_s.at[pl.ds(sid * RPR, RPR)])

        plsc.subcore_barrier()

        def chunk(j, carry):
            pltpu.async_copy(ent_h.at[src_v.at[j]], rows_v, sem).wait()
            pltpu.sync_copy(rows_v, sums_s.at[rel_v.at[j]], add=True)
            pltpu.sync_copy(ones_v, cnt_s.at[rel_v.at[j]], add=True)
            pltpu.sync_copy(ones_v, deg_s.at[dst_v.at[j]], add=True)
            return carry

        def sblock(s, carry):
            pltpu.sync_copy(src_h.at[wid, pl.ds(s * SUP, SUP)], src_v)
            pltpu.sync_copy(dst_h.at[wid, pl.ds(s * SUP, SUP)], dst_v)
            pltpu.sync_copy(rel_h.at[wid, pl.ds(s * SUP, SUP)], rel_v)
            return lax.fori_loop(0, SUP, chunk, carry)

        lax.fori_loop(0, NSB, sblock, 0)
        plsc.subcore_barrier()

        pltpu.sync_copy(deg_s.at[pl.ds(sid * RPE, RPE)],
                        deg_o.at[cid, pl.ds(sid * RPE, RPE)])
        pltpu.sync_copy(sums_s.at[pl.ds(sid * RPR, RPR)],
                        sums_o.at[cid, pl.ds(sid * RPR, RPR)])
        pltpu.sync_copy(cnt_s.at[pl.ds(sid * RPR, RPR)],
                        cnt_o.at[cid, pl.ds(sid * RPR, RPR)])

    return k(src_r, dst_r, rel_r, ent_e, zsum, zcnt, zdeg, ones16)


def _edge_agg(src_r, dst_r, rel_r, hw, relw, zagg):
    mesh = plsc.VectorSubcoreMesh(**_MESH)

    @functools.partial(
        pl.kernel,
        out_type=jax.ShapeDtypeStruct((NC, NPE, H), jnp.float32),
        mesh=mesh,
        scratch_types=[
            pltpu.VMEM((SUP, CHUNK), jnp.int32),
            pltpu.VMEM((SUP, CHUNK), jnp.int32),
            pltpu.VMEM((SUP, CHUNK), jnp.int32),
            pltpu.VMEM((CHUNK, H), jnp.float32),
            pltpu.VMEM((CHUNK, H), jnp.float32),
            pltpu.VMEM_SHARED((NPE, H), jnp.float32),
            pltpu.SemaphoreType.DMA,
            pltpu.SemaphoreType.DMA,
        ],
    )
    def k(src_h, dst_h, rel_h, hw_h, relw_h, za_h, agg_o,
          src_v, dst_v, rel_v, rows_v, rrows_v, agg_s, sem, sem2):
        cid = lax.axis_index("c")
        sid = lax.axis_index("s")
        wid = cid * NS + sid
        pltpu.sync_copy(za_h.at[pl.ds(sid * RPE, RPE)],
                        agg_s.at[pl.ds(sid * RPE, RPE)])
        plsc.subcore_barrier()

        def chunk(j, carry):
            c1 = pltpu.async_copy(hw_h.at[src_v.at[j]], rows_v, sem)
            c2 = pltpu.async_copy(relw_h.at[rel_v.at[j]], rrows_v, sem2)
            c1.wait()
            c2.wait()
            pltpu.sync_copy(rows_v, agg_s.at[dst_v.at[j]], add=True)
            pltpu.sync_copy(rrows_v, agg_s.at[dst_v.at[j]], add=True)
            return carry

        def sblock(s, carry):
            pltpu.sync_copy(src_h.at[wid, pl.ds(s * SUP, SUP)], src_v)
            pltpu.sync_copy(dst_h.at[wid, pl.ds(s * SUP, SUP)], dst_v)
            pltpu.sync_copy(rel_h.at[wid, pl.ds(s * SUP, SUP)], rel_v)
            return lax.fori_loop(0, SUP, chunk, carry)

        lax.fori_loop(0, NSB, sblock, 0)
        plsc.subcore_barrier()
        pltpu.sync_copy(agg_s.at[pl.ds(sid * RPE, RPE)],
                        agg_o.at[cid, pl.ds(sid * RPE, RPE)])

    return k(src_r, dst_r, rel_r, hw, relw, zagg)


def _gather3(ent_f, n_rel, subj, obj, rel):
    mesh = plsc.VectorSubcoreMesh(**_MESH)

    @functools.partial(
        pl.kernel,
        out_type=[
            jax.ShapeDtypeStruct((B, H), jnp.float32),
            jax.ShapeDtypeStruct((B, H), jnp.float32),
            jax.ShapeDtypeStruct((B, H), jnp.float32),
        ],
        mesh=mesh,
        scratch_types=[
            pltpu.VMEM((BPT,), jnp.int32),
            pltpu.VMEM((BPT, H), jnp.float32),
            pltpu.SemaphoreType.DMA,
        ],
    )
    def k(entf_h, nrel_h, subj_h, obj_h, rel_h, e1_o, e2_o, rg_o,
          idx_v, rows_v, sem):
        cid = lax.axis_index("c")
        sid = lax.axis_index("s")
        wid = cid * NS + sid
        base = wid * BPT
        pltpu.sync_copy(subj_h.at[pl.ds(base, BPT)], idx_v)
        pltpu.async_copy(entf_h.at[idx_v], rows_v, sem).wait()
        pltpu.sync_copy(rows_v, e1_o.at[pl.ds(base, BPT)])
        pltpu.sync_copy(obj_h.at[pl.ds(base, BPT)], idx_v)
        pltpu.async_copy(entf_h.at[idx_v], rows_v, sem).wait()
        pltpu.sync_copy(rows_v, e2_o.at[pl.ds(base, BPT)])
        pltpu.sync_copy(rel_h.at[pl.ds(base, BPT)], idx_v)
        pltpu.async_copy(nrel_h.at[idx_v], rows_v, sem).wait()
        pltpu.sync_copy(rows_v, rg_o.at[pl.ds(base, BPT)])

    return k(ent_f, n_rel, subj, obj, rel)


def _edge_stats_xla(src_r, dst_r, rel_r, ent_e, zsum, zcnt, zdeg, ones16):
    sums, cnt, deg = [], [], []
    for c in range(NC):
        s = src_r[c * NS:(c + 1) * NS].reshape(-1)
        d = dst_r[c * NS:(c + 1) * NS].reshape(-1)
        r = rel_r[c * NS:(c + 1) * NS].reshape(-1)
        sums.append(jax.ops.segment_sum(ent_e[s], r, num_segments=NPR))
        o = jnp.ones((s.shape[0], H), jnp.float32)
        cnt.append(jax.ops.segment_sum(o, r, num_segments=NPR))
        deg.append(jax.ops.segment_sum(o, d, num_segments=NPE))
    return jnp.stack(sums), jnp.stack(cnt), jnp.stack(deg)


def _edge_agg_xla(src_r, dst_r, rel_r, hw, relw, zagg):
    out = []
    for c in range(NC):
        s = src_r[c * NS:(c + 1) * NS].reshape(-1)
        d = dst_r[c * NS:(c + 1) * NS].reshape(-1)
        r = rel_r[c * NS:(c + 1) * NS].reshape(-1)
        out.append(jax.ops.segment_sum(hw[s] + relw[r], d, num_segments=NPE))
    return jnp.stack(out)


def _gather3_xla(ent_f, n_rel, subj, obj, rel):
    return ent_f[subj], ent_f[obj], n_rel[rel]


def _decoder_xla(e1, e2, convw_r, convb, fcw3t, fcb, table, n_out):
    zc = jnp.zeros((B, 1), jnp.float32)
    xs = []
    for x in (e1, e2):
        xs.append([jnp.concatenate([zc, x[:, :H - 1]], axis=1), x,
                   jnp.concatenate([x[:, 1:], zc], axis=1)])
    y = (xs[0][0][:, None, :] * convw_r[0, 0][None, :, None]
         + xs[0][1][:, None, :] * convw_r[0, 1][None, :, None]
         + xs[0][2][:, None, :] * convw_r[0, 2][None, :, None]
         + xs[1][0][:, None, :] * convw_r[1, 0][None, :, None]
         + xs[1][1][:, None, :] * convw_r[1, 1][None, :, None]
         + xs[1][2][:, None, :] * convw_r[1, 2][None, :, None]
         + convb[None, :, None])
    y = jnp.maximum(y, 0.0)
    t = jnp.einsum('bcl,clj->bj', y, fcw3t)
    t = jnp.maximum(t + fcb, 0.0)
    return t @ table.T


_RB = 2000
_NRB = N_ENT // _RB


def _full(shape):
    nd = len(shape)
    return pl.BlockSpec(shape, lambda i: (0,) * nd)


def _rows(w):
    return pl.BlockSpec((_RB, w), lambda i: (i, 0))


def _tc_l2norm(x):
    def body(x_ref, o_ref):
        o_ref[...] = _l2n(x_ref[...])

    return pl.pallas_call(
        body,
        grid=(_NRB,),
        in_specs=[_rows(H)],
        out_specs=_rows(H),
        out_shape=jax.ShapeDtypeStruct((N_ENT, H), jnp.float32),
    )(x)


def _tc_relgru(rel_embeds, sums_p, cnt_p, w_ih, w_hh, b_ih, b_hh, wr1, wr2):
    def body(re_ref, sums_ref, cnt_ref, wih_ref, whh_ref, bih_ref, bhh_ref,
             wr1_ref, wr2_ref, nrel_ref, rw1_ref, rw2_ref):
        rel_emb = re_ref[...]
        rel_e = _l2n(rel_emb)
        sums = sums_ref[0] + sums_ref[1]
        cnts = cnt_ref[0, :, 0] + cnt_ref[1, :, 0]
        rel_ent = jnp.where(cnts[:, None] > 0,
                            sums / jnp.maximum(cnts, 1.0)[:, None], 0.0)
        r_rel = jnp.concatenate([rel_emb, rel_ent], axis=-1)
        dn = (((1,), (1,)), ((), ()))
        gi = lax.dot_general(r_rel, wih_ref[...], dn,
                             preferred_element_type=jnp.float32) + bih_ref[...]
        gh = lax.dot_general(rel_e, whh_ref[...], dn,
                             preferred_element_type=jnp.float32) + bhh_ref[...]
        i_r, i_z, i_n = gi[:, :H], gi[:, H:2 * H], gi[:, 2 * H:]
        h_r, h_z, h_n = gh[:, :H], gh[:, H:2 * H], gh[:, 2 * H:]
        r = jax.nn.sigmoid(i_r + h_r)
        z = jax.nn.sigmoid(i_z + h_z)
        n = jnp.tanh(i_n + r * h_n)
        n_rel = _l2n((1.0 - z) * n + z * rel_e)
        nrel_ref[...] = n_rel
        rw1_ref[...] = lax.dot_general(n_rel, wr1_ref[...], dn,
                                       preferred_element_type=jnp.float32)
        rw2_ref[...] = lax.dot_general(n_rel, wr2_ref[...], dn,
                                       preferred_element_type=jnp.float32)

    return pl.pallas_call(
        body,
        grid=(1,),
        in_specs=[_full((N_REL, H)), _full((NC, N_REL, H)),
                  _full((NC, N_REL, H)), _full((3 * H, 2 * H)),
                  _full((3 * H, H)), _full((3 * H,)), _full((3 * H,)),
                  _full((H, H)), _full((H, H))],
        out_specs=[_full((N_REL, H))] * 3,
        out_shape=[jax.ShapeDtypeStruct((N_REL, H), jnp.float32)] * 3,
    )(rel_embeds, sums_p, cnt_p, w_ih, w_hh, b_ih, b_hh, wr1, wr2)


def _tc_entprep(ent_e, deg0, deg1, wr1, wsl1, wel1, lin_w, lin_b):
    def body(e_ref, d0_ref, d1_ref, wr_ref, wsl_ref, wel_ref, lw_ref, lb_ref,
             hw_ref, s_ref, u_ref):
        e = e_ref[...]
        deg = d0_ref[:, :1] + d1_ref[:, :1]
        dn = (((1,), (1,)), ((), ()))
        hw_ref[...] = lax.dot_general(e, wr_ref[...], dn,
                                      preferred_element_type=jnp.float32)
        msl = lax.dot_general(e, wsl_ref[...], dn,
                              preferred_element_type=jnp.float32)
        mel = lax.dot_general(e, wel_ref[...], dn,
                              preferred_element_type=jnp.float32)
        s_ref[...] = jnp.where(deg == 0.0, mel, msl)
        u_ref[...] = jax.nn.sigmoid(
            lax.dot_general(e, lw_ref[...], dn,
                            preferred_element_type=jnp.float32) + lb_ref[...])

    return pl.pallas_call(
        body,
        grid=(_NRB,),
        in_specs=[_rows(H), _rows(H), _rows(H), _full((H, H)),
                  _full((H, H)), _full((H, H)), _full((H, H)), _full((H,))],
        out_specs=[_rows(H)] * 3,
        out_shape=[jax.ShapeDtypeStruct((N_ENT, H), jnp.float32)] * 3,
    )(ent_e, deg0, deg1, wr1, wsl1, wel1, lin_w, lin_b)


def _tc_layer2prep(agg0, agg1, self1, deg0, deg1, wr2, wsl2, wel2):
    def body(a0_ref, a1_ref, s1_ref, d0_ref, d1_ref, wr_ref, wsl_ref, wel_ref,
             hw_ref, s2_ref):
        deg = d0_ref[:, :1] + d1_ref[:, :1]
        norm = 1.0 / jnp.maximum(deg, 1.0)
        x = (a0_ref[...] + a1_ref[...]) * norm + s1_ref[...]
        h1 = _lrelu(x)
        dn = (((1,), (1,)), ((), ()))
        hw_ref[...] = lax.dot_general(h1, wr_ref[...], dn,
                                      preferred_element_type=jnp.float32)
        msl = lax.dot_general(h1, wsl_ref[...], dn,
                              preferred_element_type=jnp.float32)
        mel = lax.dot_general(h1, wel_ref[...], dn,
                              preferred_element_type=jnp.float32)
        s2_ref[...] = jnp.where(deg == 0.0, mel, msl)

    return pl.pallas_call(
        body,
        grid=(_NRB,),
        in_specs=[_rows(H), _rows(H), _rows(H), _rows(H), _rows(H),
                  _full((H, H)), _full((H, H)), _full((H, H))],
        out_specs=[_rows(H)] * 2,
        out_shape=[jax.ShapeDtypeStruct((N_ENT, H), jnp.float32)] * 2,
    )(agg0, agg1, self1, deg0, deg1, wr2, wsl2, wel2)


def _tc_entfinal(agg0, agg1, self2, deg0, deg1, ent_e, u):
    def body(a0_ref, a1_ref, s2_ref, d0_ref, d1_ref, e_ref, u_ref, o_ref):
        deg = d0_ref[:, :1] + d1_ref[:, :1]
        norm = 1.0 / jnp.maximum(deg, 1.0)
        x = (a0_ref[...] + a1_ref[...]) * norm + s2_ref[...]
        w_ent = _l2n(_lrelu(x))
        e = e_ref[...]
        o_ref[...] = e + u_ref[...] * (w_ent - e)

    return pl.pallas_call(
        body,
        grid=(_NRB,),
        in_specs=[_rows(H)] * 3 + [_rows(H), _rows(H), _rows(H), _rows(H)],
        out_specs=_rows(H),
        out_shape=jax.ShapeDtypeStruct((N_ENT, H), jnp.float32),
    )(agg0, agg1, self2, deg0, deg1, ent_e, u)


_BB = 128


def _tc_dec_feat(e1, e2, convw_r, convb, fcw_t, fcb):
    def body(e1_ref, e2_ref, cw_ref, cb_ref, fw_ref, fb_ref, t_ref):
        x1 = e1_ref[...]
        x2 = e2_ref[...]
        z = jnp.zeros((_BB, 1), jnp.float32)
        x1m = jnp.concatenate([z, x1[:, :H - 1]], axis=1)
        x1p = jnp.concatenate([x1[:, 1:], z], axis=1)
        x2m = jnp.concatenate([z, x2[:, :H - 1]], axis=1)
        x2p = jnp.concatenate([x2[:, 1:], z], axis=1)
        cw = cw_ref[...]
        cb = cb_ref[...]
        y = (x1m[:, None, :] * cw[0, 0][None, :, None]
             + x1[:, None, :] * cw[0, 1][None, :, None]
             + x1p[:, None, :] * cw[0, 2][None, :, None]
             + x2m[:, None, :] * cw[1, 0][None, :, None]
             + x2[:, None, :] * cw[1, 1][None, :, None]
             + x2p[:, None, :] * cw[1, 2][None, :, None]
             + cb[None, :, None])
        y = jnp.maximum(y, 0.0).reshape(_BB, CH * H)
        t_ref[...] = jnp.maximum(
            jnp.dot(y, fw_ref[...], preferred_element_type=jnp.float32)
            + fb_ref[...], 0.0)

    return pl.pallas_call(
        body,
        grid=(B // _BB,),
        in_specs=[pl.BlockSpec((_BB, H), lambda i: (i, 0)),
                  pl.BlockSpec((_BB, H), lambda i: (i, 0)),
                  _full((2, K, CH)), _full((CH,)), _full((CH * H, H)),
                  _full((H,))],
        out_specs=pl.BlockSpec((_BB, H), lambda i: (i, 0)),
        out_shape=jax.ShapeDtypeStruct((B, H), jnp.float32),
    )(e1, e2, convw_r, convb, fcw_t, fcb)


def _tc_logits(t, table, n_pad, vb):
    def body(t_ref, tab_ref, o_ref):
        dn = (((1,), (1,)), ((), ()))
        o_ref[...] = lax.dot_general(t_ref[...], tab_ref[...], dn,
                                     preferred_element_type=jnp.float32)

    return pl.pallas_call(
        body,
        grid=(B // _BB, n_pad // vb),
        in_specs=[pl.BlockSpec((_BB, H), lambda i, j: (i, 0)),
                  pl.BlockSpec((vb, H), lambda i, j: (j, 0))],
        out_specs=pl.BlockSpec((_BB, vb), lambda i, j: (i, j)),
        out_shape=jax.ShapeDtypeStruct((B, n_pad), jnp.float32),
    )(t, table)


@jax.jit
def kernel(ent_embeds, rel_embeds, W_r1, W_sl1, W_el1, W_r2, W_sl2, W_el2,
           lin_w, lin_b, gru_w_ih, gru_w_hh, gru_b_ih, gru_b_hh, convR_w,
           convR_b, fcR_w, fcR_b, convE_w, convE_b, fcE_w, fcE_b, edge_index,
           edge_rel, subj, rel, obj):
    npad = E_PAD - E
    src_r = jnp.concatenate(
        [edge_index[0].astype(jnp.int32), jnp.zeros((npad,), jnp.int32)]
    ).reshape(NW, NCHUNK, CHUNK)
    dst_r = jnp.concatenate(
        [edge_index[1].astype(jnp.int32),
         jnp.full((npad,), N_ENT, jnp.int32)]
    ).reshape(NW, NCHUNK, CHUNK)
    rel_r = jnp.concatenate(
        [edge_rel.astype(jnp.int32), jnp.full((npad,), N_REL, jnp.int32)]
    ).reshape(NW, NCHUNK, CHUNK)
    subj_i = subj.astype(jnp.int32)
    obj_i = obj.astype(jnp.int32)
    rel_i = rel.astype(jnp.int32)

    zsum = jnp.zeros((NPR, H), jnp.float32)
    zcnt = jnp.zeros((NPR, H), jnp.float32)
    zdeg = jnp.zeros((NPE, H), jnp.float32)
    zagg = jnp.zeros((NPE, H), jnp.float32)
    ones16 = jnp.ones((CHUNK, H), jnp.float32)

    ent_e = _tc_l2norm(ent_embeds)

    sums_pp, cnt_pp, deg_pp = _edge_stats(src_r, dst_r, rel_r, ent_e,
                                          zsum, zcnt, zdeg, ones16)
    sums_p = sums_pp[:, :N_REL]
    cnt_p = cnt_pp[:, :N_REL]
    deg0 = deg_pp[0, :N_ENT]
    deg1 = deg_pp[1, :N_ENT]

    n_rel, relW1, relW2 = _tc_relgru(rel_embeds, sums_p, cnt_p, gru_w_ih,
                                     gru_w_hh, gru_b_ih, gru_b_hh, W_r1, W_r2)
    zrel = jnp.zeros((NPR - N_REL, H), jnp.float32)
    relW1 = jnp.concatenate([relW1, zrel])
    relW2 = jnp.concatenate([relW2, zrel])

    hW1, self1, u = _tc_entprep(ent_e, deg0, deg1, W_r1, W_sl1, W_el1,
                                lin_w, lin_b)

    agg1_p = _edge_agg(src_r, dst_r, rel_r, hW1, relW1, zagg)
    hW2, self2 = _tc_layer2prep(agg1_p[0, :N_ENT], agg1_p[1, :N_ENT], self1,
                                deg0, deg1, W_r2, W_sl2, W_el2)

    agg2_p = _edge_agg(src_r, dst_r, rel_r, hW2, relW2, zagg)
    ent_f = _tc_entfinal(agg2_p[0, :N_ENT], agg2_p[1, :N_ENT], self2, deg0,
                         deg1, ent_e, u)

    e1, e2, rg = _gather3(ent_f, n_rel, subj_i, obj_i, rel_i)

    convR_r = convR_w.transpose(1, 2, 0)
    convE_r = convE_w.transpose(1, 2, 0)

    tR = _tc_dec_feat(e1, e2, convR_r, convR_b, fcR_w.T, fcR_b)
    tE = _tc_dec_feat(e1, rg, convE_r, convE_b, fcE_w.T, fcE_b)

    nrel_pad = jnp.concatenate(
        [n_rel, jnp.zeros((NPR - N_REL, H), jnp.float32)])
    entf_pad = jnp.concatenate(
        [ent_f, jnp.zeros((NPE - N_ENT, H), jnp.float32)])
    rel_logit = _tc_logits(tR, nrel_pad, NPR, NPR)[:, :N_REL]
    obj_logit = _tc_logits(tE, entf_pad, NPE, 1280)[:, :N_ENT]
    return (obj_logit, rel_logit)


# speedup vs baseline: 2.6098x; 1.1490x over previous
import functools

import jax
import jax.numpy as jnp
from jax import lax
from jax.experimental import pallas as pl
from jax.experimental.pallas import tpu as pltpu
from jax.experimental.pallas import tpu_sc as plsc

N_ENT = 10000
N_REL = 200
H = 128
CH = 50
K = 3
B = 1024
E = 320000
RRELU_SLOPE = (1.0 / 8.0 + 1.0 / 3.0) / 2.0

NC = 2
NS = 16
NPE = 10240
NPR = 256
RPE = NPE // NS
RPR = NPR // NS
NW = NC * NS
CHUNK = 80
NCHUNK = 128
SUP = 8
NSB = NCHUNK // SUP
EPT = NCHUNK * CHUNK
E_PAD = NW * EPT
BPT = B // NW

_MESH = dict(core_axis_name="c", subcore_axis_name="s", num_cores=NC,
             num_subcores=NS)


def _l2n(x):
    n = jnp.sqrt(jnp.sum(x * x, axis=-1, keepdims=True))
    return x / jnp.maximum(n, 1e-12)


def _lrelu(x):
    return jnp.where(x >= 0, x, RRELU_SLOPE * x)


def _edge_stats(src_r, dst_r, rel_r, ent_e, zsum, zcnt, zdeg, ones16):
    mesh = plsc.VectorSubcoreMesh(**_MESH)

    @functools.partial(
        pl.kernel,
        out_type=[
            jax.ShapeDtypeStruct((NC, NPR, H), jnp.float32),
            jax.ShapeDtypeStruct((NC, NPR, H), jnp.float32),
            jax.ShapeDtypeStruct((NC, NPE, H), jnp.float32),
        ],
        mesh=mesh,
        scratch_types=[
            pltpu.VMEM((SUP, CHUNK), jnp.int32),
            pltpu.VMEM((SUP, CHUNK), jnp.int32),
            pltpu.VMEM((SUP, CHUNK), jnp.int32),
            pltpu.VMEM((CHUNK, H), jnp.float32),
            pltpu.VMEM((CHUNK, H), jnp.float32),
            pltpu.VMEM((CHUNK, H), jnp.float32),
            pltpu.VMEM_SHARED((NPR, H), jnp.float32),
            pltpu.VMEM_SHARED((NPR, H), jnp.float32),
            pltpu.VMEM_SHARED((NPE, H), jnp.float32),
            pltpu.SemaphoreType.DMA,
            pltpu.SemaphoreType.DMA,
        ],
    )
    def k(src_h, dst_h, rel_h, ent_h, zs_h, zc_h, zd_h, ones_h,
          sums_o, cnt_o, deg_o,
          src_v, dst_v, rel_v, rows_a, rows_b, ones_v, sums_s, cnt_s, deg_s,
          sem_a, sem_b):
        cid = lax.axis_index("c")
        sid = lax.axis_index("s")
        wid = cid * NS + sid
        pltpu.sync_copy(ones_h, ones_v)
        pltpu.sync_copy(zd_h.at[pl.ds(sid * RPE, RPE)],
                        deg_s.at[pl.ds(sid * RPE, RPE)])
        pltpu.sync_copy(zs_h.at[pl.ds(sid * RPR, RPR)],
                        sums_s.at[pl.ds(sid * RPR, RPR)])
        pltpu.sync_copy(zc_h.at[pl.ds(sid * RPR, RPR)],
                        cnt_s.at[pl.ds(sid * RPR, RPR)])

        plsc.subcore_barrier()

        bufs = ((rows_a, sem_a), (rows_b, sem_b))

        def scatter(j, rv):
            pltpu.sync_copy(rv, sums_s.at[rel_v.at[j]], add=True)
            pltpu.sync_copy(ones_v, cnt_s.at[rel_v.at[j]], add=True)
            pltpu.sync_copy(ones_v, deg_s.at[dst_v.at[j]], add=True)

        def sblock(s, carry):
            pltpu.sync_copy(src_h.at[wid, pl.ds(s * SUP, SUP)], src_v)
            pltpu.sync_copy(dst_h.at[wid, pl.ds(s * SUP, SUP)], dst_v)
            pltpu.sync_copy(rel_h.at[wid, pl.ds(s * SUP, SUP)], rel_v)
            cps = []
            for j in range(SUP):
                rv, sa = bufs[j % 2]
                cps.append(pltpu.async_copy(ent_h.at[src_v.at[j]], rv, sa))
                if j >= 1:
                    cps[j - 1].wait()
                    scatter(j - 1, bufs[(j - 1) % 2][0])
            cps[SUP - 1].wait()
            scatter(SUP - 1, bufs[(SUP - 1) % 2][0])
            return carry

        lax.fori_loop(0, NSB, sblock, 0)
        plsc.subcore_barrier()

        pltpu.sync_copy(deg_s.at[pl.ds(sid * RPE, RPE)],
                        deg_o.at[cid, pl.ds(sid * RPE, RPE)])
        pltpu.sync_copy(sums_s.at[pl.ds(sid * RPR, RPR)],
                        sums_o.at[cid, pl.ds(sid * RPR, RPR)])
        pltpu.sync_copy(cnt_s.at[pl.ds(sid * RPR, RPR)],
                        cnt_o.at[cid, pl.ds(sid * RPR, RPR)])

    return k(src_r, dst_r, rel_r, ent_e, zsum, zcnt, zdeg, ones16)


def _edge_agg(src_r, dst_r, rel_r, hw, relw, zagg):
    mesh = plsc.VectorSubcoreMesh(**_MESH)

    @functools.partial(
        pl.kernel,
        out_type=jax.ShapeDtypeStruct((NC, NPE, H), jnp.float32),
        mesh=mesh,
        scratch_types=[
            pltpu.VMEM((SUP, CHUNK), jnp.int32),
            pltpu.VMEM((SUP, CHUNK), jnp.int32),
            pltpu.VMEM((SUP, CHUNK), jnp.int32),
            pltpu.VMEM((CHUNK, H), jnp.float32),
            pltpu.VMEM((CHUNK, H), jnp.float32),
            pltpu.VMEM((CHUNK, H), jnp.float32),
            pltpu.VMEM((CHUNK, H), jnp.float32),
            pltpu.VMEM_SHARED((NPE, H), jnp.float32),
            pltpu.SemaphoreType.DMA,
            pltpu.SemaphoreType.DMA,
            pltpu.SemaphoreType.DMA,
            pltpu.SemaphoreType.DMA,
        ],
    )
    def k(src_h, dst_h, rel_h, hw_h, relw_h, za_h, agg_o,
          src_v, dst_v, rel_v, rows_a, rrows_a, rows_b, rrows_b, agg_s,
          sem_a, sem2_a, sem_b, sem2_b):
        cid = lax.axis_index("c")
        sid = lax.axis_index("s")
        wid = cid * NS + sid
        pltpu.sync_copy(za_h.at[pl.ds(sid * RPE, RPE)],
                        agg_s.at[pl.ds(sid * RPE, RPE)])
        plsc.subcore_barrier()

        bufs = ((rows_a, rrows_a, sem_a, sem2_a),
                (rows_b, rrows_b, sem_b, sem2_b))

        def sblock(s, carry):
            pltpu.sync_copy(src_h.at[wid, pl.ds(s * SUP, SUP)], src_v)
            pltpu.sync_copy(dst_h.at[wid, pl.ds(s * SUP, SUP)], dst_v)
            pltpu.sync_copy(rel_h.at[wid, pl.ds(s * SUP, SUP)], rel_v)
            cps = []
            for j in range(SUP):
                rv, rr, sa, sb = bufs[j % 2]
                cps.append((pltpu.async_copy(hw_h.at[src_v.at[j]], rv, sa),
                            pltpu.async_copy(relw_h.at[rel_v.at[j]], rr, sb)))
                if j >= 1:
                    jj = j - 1
                    pv, pr, _, _ = bufs[jj % 2]
                    cps[jj][0].wait()
                    cps[jj][1].wait()
                    pltpu.sync_copy(pv, agg_s.at[dst_v.at[jj]], add=True)
                    pltpu.sync_copy(pr, agg_s.at[dst_v.at[jj]], add=True)
            jj = SUP - 1
            pv, pr, _, _ = bufs[jj % 2]
            cps[jj][0].wait()
            cps[jj][1].wait()
            pltpu.sync_copy(pv, agg_s.at[dst_v.at[jj]], add=True)
            pltpu.sync_copy(pr, agg_s.at[dst_v.at[jj]], add=True)
            return carry

        lax.fori_loop(0, NSB, sblock, 0)
        plsc.subcore_barrier()
        pltpu.sync_copy(agg_s.at[pl.ds(sid * RPE, RPE)],
                        agg_o.at[cid, pl.ds(sid * RPE, RPE)])

    return k(src_r, dst_r, rel_r, hw, relw, zagg)


def _gather3(ent_f, n_rel, subj, obj, rel):
    mesh = plsc.VectorSubcoreMesh(**_MESH)

    @functools.partial(
        pl.kernel,
        out_type=[
            jax.ShapeDtypeStruct((B, H), jnp.float32),
            jax.ShapeDtypeStruct((B, H), jnp.float32),
            jax.ShapeDtypeStruct((B, H), jnp.float32),
        ],
        mesh=mesh,
        scratch_types=[
            pltpu.VMEM((BPT,), jnp.int32),
            pltpu.VMEM((BPT, H), jnp.float32),
            pltpu.SemaphoreType.DMA,
        ],
    )
    def k(entf_h, nrel_h, subj_h, obj_h, rel_h, e1_o, e2_o, rg_o,
          idx_v, rows_v, sem):
        cid = lax.axis_index("c")
        sid = lax.axis_index("s")
        wid = cid * NS + sid
        base = wid * BPT
        pltpu.sync_copy(subj_h.at[pl.ds(base, BPT)], idx_v)
        pltpu.async_copy(entf_h.at[idx_v], rows_v, sem).wait()
        pltpu.sync_copy(rows_v, e1_o.at[pl.ds(base, BPT)])
        pltpu.sync_copy(obj_h.at[pl.ds(base, BPT)], idx_v)
        pltpu.async_copy(entf_h.at[idx_v], rows_v, sem).wait()
        pltpu.sync_copy(rows_v, e2_o.at[pl.ds(base, BPT)])
        pltpu.sync_copy(rel_h.at[pl.ds(base, BPT)], idx_v)
        pltpu.async_copy(nrel_h.at[idx_v], rows_v, sem).wait()
        pltpu.sync_copy(rows_v, rg_o.at[pl.ds(base, BPT)])

    return k(ent_f, n_rel, subj, obj, rel)


def _edge_stats_xla(src_r, dst_r, rel_r, ent_e, zsum, zcnt, zdeg, ones16):
    sums, cnt, deg = [], [], []
    for c in range(NC):
        s = src_r[c * NS:(c + 1) * NS].reshape(-1)
        d = dst_r[c * NS:(c + 1) * NS].reshape(-1)
        r = rel_r[c * NS:(c + 1) * NS].reshape(-1)
        sums.append(jax.ops.segment_sum(ent_e[s], r, num_segments=NPR))
        o = jnp.ones((s.shape[0], H), jnp.float32)
        cnt.append(jax.ops.segment_sum(o, r, num_segments=NPR))
        deg.append(jax.ops.segment_sum(o, d, num_segments=NPE))
    return jnp.stack(sums), jnp.stack(cnt), jnp.stack(deg)


def _edge_agg_xla(src_r, dst_r, rel_r, hw, relw, zagg):
    out = []
    for c in range(NC):
        s = src_r[c * NS:(c + 1) * NS].reshape(-1)
        d = dst_r[c * NS:(c + 1) * NS].reshape(-1)
        r = rel_r[c * NS:(c + 1) * NS].reshape(-1)
        out.append(jax.ops.segment_sum(hw[s] + relw[r], d, num_segments=NPE))
    return jnp.stack(out)


def _gather3_xla(ent_f, n_rel, subj, obj, rel):
    return ent_f[subj], ent_f[obj], n_rel[rel]


def _decoder_xla(e1, e2, convw_r, convb, fcw3t, fcb, table, n_out):
    zc = jnp.zeros((B, 1), jnp.float32)
    xs = []
    for x in (e1, e2):
        xs.append([jnp.concatenate([zc, x[:, :H - 1]], axis=1), x,
                   jnp.concatenate([x[:, 1:], zc], axis=1)])
    y = (xs[0][0][:, None, :] * convw_r[0, 0][None, :, None]
         + xs[0][1][:, None, :] * convw_r[0, 1][None, :, None]
         + xs[0][2][:, None, :] * convw_r[0, 2][None, :, None]
         + xs[1][0][:, None, :] * convw_r[1, 0][None, :, None]
         + xs[1][1][:, None, :] * convw_r[1, 1][None, :, None]
         + xs[1][2][:, None, :] * convw_r[1, 2][None, :, None]
         + convb[None, :, None])
    y = jnp.maximum(y, 0.0)
    t = jnp.einsum('bcl,clj->bj', y, fcw3t)
    t = jnp.maximum(t + fcb, 0.0)
    return t @ table.T


_RB = 2000
_NRB = N_ENT // _RB


def _full(shape):
    nd = len(shape)
    return pl.BlockSpec(shape, lambda i: (0,) * nd)


def _rows(w):
    return pl.BlockSpec((_RB, w), lambda i: (i, 0))


def _tc_l2norm(x):
    def body(x_ref, o_ref):
        o_ref[...] = _l2n(x_ref[...])

    return pl.pallas_call(
        body,
        grid=(_NRB,),
        in_specs=[_rows(H)],
        out_specs=_rows(H),
        out_shape=jax.ShapeDtypeStruct((N_ENT, H), jnp.float32),
    )(x)


def _tc_relgru(rel_embeds, sums_p, cnt_p, w_ih, w_hh, b_ih, b_hh, wr1, wr2):
    def body(re_ref, sums_ref, cnt_ref, wih_ref, whh_ref, bih_ref, bhh_ref,
             wr1_ref, wr2_ref, nrel_ref, rw1_ref, rw2_ref):
        rel_emb = re_ref[...]
        rel_e = _l2n(rel_emb)
        sums = sums_ref[0] + sums_ref[1]
        cnts = cnt_ref[0, :, 0] + cnt_ref[1, :, 0]
        rel_ent = jnp.where(cnts[:, None] > 0,
                            sums / jnp.maximum(cnts, 1.0)[:, None], 0.0)
        r_rel = jnp.concatenate([rel_emb, rel_ent], axis=-1)
        dn = (((1,), (1,)), ((), ()))
        gi = lax.dot_general(r_rel, wih_ref[...], dn,
                             preferred_element_type=jnp.float32) + bih_ref[...]
        gh = lax.dot_general(rel_e, whh_ref[...], dn,
                             preferred_element_type=jnp.float32) + bhh_ref[...]
        i_r, i_z, i_n = gi[:, :H], gi[:, H:2 * H], gi[:, 2 * H:]
        h_r, h_z, h_n = gh[:, :H], gh[:, H:2 * H], gh[:, 2 * H:]
        r = jax.nn.sigmoid(i_r + h_r)
        z = jax.nn.sigmoid(i_z + h_z)
        n = jnp.tanh(i_n + r * h_n)
        n_rel = _l2n((1.0 - z) * n + z * rel_e)
        nrel_ref[...] = n_rel
        rw1_ref[...] = lax.dot_general(n_rel, wr1_ref[...], dn,
                                       preferred_element_type=jnp.float32)
        rw2_ref[...] = lax.dot_general(n_rel, wr2_ref[...], dn,
                                       preferred_element_type=jnp.float32)

    return pl.pallas_call(
        body,
        grid=(1,),
        in_specs=[_full((N_REL, H)), _full((NC, N_REL, H)),
                  _full((NC, N_REL, H)), _full((3 * H, 2 * H)),
                  _full((3 * H, H)), _full((3 * H,)), _full((3 * H,)),
                  _full((H, H)), _full((H, H))],
        out_specs=[_full((N_REL, H))] * 3,
        out_shape=[jax.ShapeDtypeStruct((N_REL, H), jnp.float32)] * 3,
    )(rel_embeds, sums_p, cnt_p, w_ih, w_hh, b_ih, b_hh, wr1, wr2)


def _tc_entprep(ent_e, deg0, deg1, wr1, wsl1, wel1, lin_w, lin_b):
    def body(e_ref, d0_ref, d1_ref, wr_ref, wsl_ref, wel_ref, lw_ref, lb_ref,
             hw_ref, s_ref, u_ref):
        e = e_ref[...]
        deg = d0_ref[:, :1] + d1_ref[:, :1]
        dn = (((1,), (1,)), ((), ()))
        hw_ref[...] = lax.dot_general(e, wr_ref[...], dn,
                                      preferred_element_type=jnp.float32)
        msl = lax.dot_general(e, wsl_ref[...], dn,
                              preferred_element_type=jnp.float32)
        mel = lax.dot_general(e, wel_ref[...], dn,
                              preferred_element_type=jnp.float32)
        s_ref[...] = jnp.where(deg == 0.0, mel, msl)
        u_ref[...] = jax.nn.sigmoid(
            lax.dot_general(e, lw_ref[...], dn,
                            preferred_element_type=jnp.float32) + lb_ref[...])

    return pl.pallas_call(
        body,
        grid=(_NRB,),
        in_specs=[_rows(H), _rows(H), _rows(H), _full((H, H)),
                  _full((H, H)), _full((H, H)), _full((H, H)), _full((H,))],
        out_specs=[_rows(H)] * 3,
        out_shape=[jax.ShapeDtypeStruct((N_ENT, H), jnp.float32)] * 3,
    )(ent_e, deg0, deg1, wr1, wsl1, wel1, lin_w, lin_b)


def _tc_layer2prep(agg0, agg1, self1, deg0, deg1, wr2, wsl2, wel2):
    def body(a0_ref, a1_ref, s1_ref, d0_ref, d1_ref, wr_ref, wsl_ref, wel_ref,
             hw_ref, s2_ref):
        deg = d0_ref[:, :1] + d1_ref[:, :1]
        norm = 1.0 / jnp.maximum(deg, 1.0)
        x = (a0_ref[...] + a1_ref[...]) * norm + s1_ref[...]
        h1 = _lrelu(x)
        dn = (((1,), (1,)), ((), ()))
        hw_ref[...] = lax.dot_general(h1, wr_ref[...], dn,
                                      preferred_element_type=jnp.float32)
        msl = lax.dot_general(h1, wsl_ref[...], dn,
                              preferred_element_type=jnp.float32)
        mel = lax.dot_general(h1, wel_ref[...], dn,
                              preferred_element_type=jnp.float32)
        s2_ref[...] = jnp.where(deg == 0.0, mel, msl)

    return pl.pallas_call(
        body,
        grid=(_NRB,),
        in_specs=[_rows(H), _rows(H), _rows(H), _rows(H), _rows(H),
                  _full((H, H)), _full((H, H)), _full((H, H))],
        out_specs=[_rows(H)] * 2,
        out_shape=[jax.ShapeDtypeStruct((N_ENT, H), jnp.float32)] * 2,
    )(agg0, agg1, self1, deg0, deg1, wr2, wsl2, wel2)


def _tc_entfinal(agg0, agg1, self2, deg0, deg1, ent_e, u):
    def body(a0_ref, a1_ref, s2_ref, d0_ref, d1_ref, e_ref, u_ref, o_ref):
        deg = d0_ref[:, :1] + d1_ref[:, :1]
        norm = 1.0 / jnp.maximum(deg, 1.0)
        x = (a0_ref[...] + a1_ref[...]) * norm + s2_ref[...]
        w_ent = _l2n(_lrelu(x))
        e = e_ref[...]
        o_ref[...] = e + u_ref[...] * (w_ent - e)

    return pl.pallas_call(
        body,
        grid=(_NRB,),
        in_specs=[_rows(H)] * 3 + [_rows(H), _rows(H), _rows(H), _rows(H)],
        out_specs=_rows(H),
        out_shape=jax.ShapeDtypeStruct((N_ENT, H), jnp.float32),
    )(agg0, agg1, self2, deg0, deg1, ent_e, u)


_BB = 128


def _tc_dec_feat(e1, e2, convw_r, convb, fcw_t, fcb):
    def body(e1_ref, e2_ref, cw_ref, cb_ref, fw_ref, fb_ref, t_ref):
        x1 = e1_ref[...]
        x2 = e2_ref[...]
        z = jnp.zeros((_BB, 1), jnp.float32)
        x1m = jnp.concatenate([z, x1[:, :H - 1]], axis=1)
        x1p = jnp.concatenate([x1[:, 1:], z], axis=1)
        x2m = jnp.concatenate([z, x2[:, :H - 1]], axis=1)
        x2p = jnp.concatenate([x2[:, 1:], z], axis=1)
        cw = cw_ref[...]
        cb = cb_ref[...]
        y = (x1m[:, None, :] * cw[0, 0][None, :, None]
             + x1[:, None, :] * cw[0, 1][None, :, None]
             + x1p[:, None, :] * cw[0, 2][None, :, None]
             + x2m[:, None, :] * cw[1, 0][None, :, None]
             + x2[:, None, :] * cw[1, 1][None, :, None]
             + x2p[:, None, :] * cw[1, 2][None, :, None]
             + cb[None, :, None])
        y = jnp.maximum(y, 0.0).reshape(_BB, CH * H)
        t_ref[...] = jnp.maximum(
            jnp.dot(y, fw_ref[...], preferred_element_type=jnp.float32)
            + fb_ref[...], 0.0)

    return pl.pallas_call(
        body,
        grid=(B // _BB,),
        in_specs=[pl.BlockSpec((_BB, H), lambda i: (i, 0)),
                  pl.BlockSpec((_BB, H), lambda i: (i, 0)),
                  _full((2, K, CH)), _full((CH,)), _full((CH * H, H)),
                  _full((H,))],
        out_specs=pl.BlockSpec((_BB, H), lambda i: (i, 0)),
        out_shape=jax.ShapeDtypeStruct((B, H), jnp.float32),
    )(e1, e2, convw_r, convb, fcw_t, fcb)


def _tc_logits(t, table, n_pad, vb):
    def body(t_ref, tab_ref, o_ref):
        dn = (((1,), (1,)), ((), ()))
        o_ref[...] = lax.dot_general(t_ref[...], tab_ref[...], dn,
                                     preferred_element_type=jnp.float32)

    return pl.pallas_call(
        body,
        grid=(B // _BB, n_pad // vb),
        in_specs=[pl.BlockSpec((_BB, H), lambda i, j: (i, 0)),
                  pl.BlockSpec((vb, H), lambda i, j: (j, 0))],
        out_specs=pl.BlockSpec((_BB, vb), lambda i, j: (i, j)),
        out_shape=jax.ShapeDtypeStruct((B, n_pad), jnp.float32),
    )(t, table)


@jax.jit
def kernel(ent_embeds, rel_embeds, W_r1, W_sl1, W_el1, W_r2, W_sl2, W_el2,
           lin_w, lin_b, gru_w_ih, gru_w_hh, gru_b_ih, gru_b_hh, convR_w,
           convR_b, fcR_w, fcR_b, convE_w, convE_b, fcE_w, fcE_b, edge_index,
           edge_rel, subj, rel, obj):
    npad = E_PAD - E
    src_r = jnp.concatenate(
        [edge_index[0].astype(jnp.int32), jnp.zeros((npad,), jnp.int32)]
    ).reshape(NW, NCHUNK, CHUNK)
    dst_r = jnp.concatenate(
        [edge_index[1].astype(jnp.int32),
         jnp.full((npad,), N_ENT, jnp.int32)]
    ).reshape(NW, NCHUNK, CHUNK)
    rel_r = jnp.concatenate(
        [edge_rel.astype(jnp.int32), jnp.full((npad,), N_REL, jnp.int32)]
    ).reshape(NW, NCHUNK, CHUNK)
    subj_i = subj.astype(jnp.int32)
    obj_i = obj.astype(jnp.int32)
    rel_i = rel.astype(jnp.int32)

    zsum = jnp.zeros((NPR, H), jnp.float32)
    zcnt = jnp.zeros((NPR, H), jnp.float32)
    zdeg = jnp.zeros((NPE, H), jnp.float32)
    zagg = jnp.zeros((NPE, H), jnp.float32)
    ones16 = jnp.ones((CHUNK, H), jnp.float32)

    ent_e = _tc_l2norm(ent_embeds)

    sums_pp, cnt_pp, deg_pp = _edge_stats(src_r, dst_r, rel_r, ent_e,
                                          zsum, zcnt, zdeg, ones16)
    sums_p = sums_pp[:, :N_REL]
    cnt_p = cnt_pp[:, :N_REL]
    deg0 = deg_pp[0, :N_ENT]
    deg1 = deg_pp[1, :N_ENT]

    n_rel, relW1, relW2 = _tc_relgru(rel_embeds, sums_p, cnt_p, gru_w_ih,
                                     gru_w_hh, gru_b_ih, gru_b_hh, W_r1, W_r2)
    zrel = jnp.zeros((NPR - N_REL, H), jnp.float32)
    relW1 = jnp.concatenate([relW1, zrel])
    relW2 = jnp.concatenate([relW2, zrel])

    hW1, self1, u = _tc_entprep(ent_e, deg0, deg1, W_r1, W_sl1, W_el1,
                                lin_w, lin_b)

    agg1_p = _edge_agg(src_r, dst_r, rel_r, hW1, relW1, zagg)
    hW2, self2 = _tc_layer2prep(agg1_p[0, :N_ENT], agg1_p[1, :N_ENT], self1,
                                deg0, deg1, W_r2, W_sl2, W_el2)

    agg2_p = _edge_agg(src_r, dst_r, rel_r, hW2, relW2, zagg)
    ent_f = _tc_entfinal(agg2_p[0, :N_ENT], agg2_p[1, :N_ENT], self2, deg0,
                         deg1, ent_e, u)

    e1, e2, rg = _gather3(ent_f, n_rel, subj_i, obj_i, rel_i)

    convR_r = convR_w.transpose(1, 2, 0)
    convE_r = convE_w.transpose(1, 2, 0)

    tR = _tc_dec_feat(e1, e2, convR_r, convR_b, fcR_w.T, fcR_b)
    tE = _tc_dec_feat(e1, rg, convE_r, convE_b, fcE_w.T, fcE_b)

    nrel_pad = jnp.concatenate(
        [n_rel, jnp.zeros((NPR - N_REL, H), jnp.float32)])
    entf_pad = jnp.concatenate(
        [ent_f, jnp.zeros((NPE - N_ENT, H), jnp.float32)])
    rel_logit = _tc_logits(tR, nrel_pad, NPR, NPR)[:, :N_REL]
    obj_logit = _tc_logits(tE, entf_pad, NPE, 1280)[:, :N_ENT]
    return (obj_logit, rel_logit)


# speedup vs baseline: 2.6144x; 1.0018x over previous
import functools

import jax
import jax.numpy as jnp
from jax import lax
from jax.experimental import pallas as pl
from jax.experimental.pallas import tpu as pltpu
from jax.experimental.pallas import tpu_sc as plsc

N_ENT = 10000
N_REL = 200
H = 128
CH = 50
K = 3
B = 1024
E = 320000
RRELU_SLOPE = (1.0 / 8.0 + 1.0 / 3.0) / 2.0

NC = 2
NS = 16
NPE = 10240
NPR = 256
RPE = NPE // NS
RPR = NPR // NS
NW = NC * NS
CHUNK = 80
NCHUNK = 128
SUP = 8
NSB = NCHUNK // SUP
EPT = NCHUNK * CHUNK
E_PAD = NW * EPT
BPT = B // NW

_MESH = dict(core_axis_name="c", subcore_axis_name="s", num_cores=NC,
             num_subcores=NS)


def _l2n(x):
    n = jnp.sqrt(jnp.sum(x * x, axis=-1, keepdims=True))
    return x / jnp.maximum(n, 1e-12)


def _lrelu(x):
    return jnp.where(x >= 0, x, RRELU_SLOPE * x)


def _edge_stats(src_r, dst_r, rel_r, ent_e, zsum, zcnt, zdeg, ones16):
    mesh = plsc.VectorSubcoreMesh(**_MESH)

    @functools.partial(
        pl.kernel,
        out_type=[
            jax.ShapeDtypeStruct((NC, NPR, H), jnp.float32),
            jax.ShapeDtypeStruct((NC, NPR, H), jnp.float32),
            jax.ShapeDtypeStruct((NC, NPE, H), jnp.float32),
        ],
        mesh=mesh,
        scratch_types=[
            pltpu.VMEM((SUP, CHUNK), jnp.int32),
            pltpu.VMEM((SUP, CHUNK), jnp.int32),
            pltpu.VMEM((SUP, CHUNK), jnp.int32),
            pltpu.VMEM((CHUNK, H), jnp.float32),
            pltpu.VMEM((CHUNK, H), jnp.float32),
            pltpu.VMEM((CHUNK, H), jnp.float32),
            pltpu.VMEM_SHARED((NPR, H), jnp.float32),
            pltpu.VMEM_SHARED((NPR, H), jnp.float32),
            pltpu.VMEM_SHARED((NPE, H), jnp.float32),
            pltpu.SemaphoreType.DMA,
            pltpu.SemaphoreType.DMA,
        ],
    )
    def k(src_h, dst_h, rel_h, ent_h, zs_h, zc_h, zd_h, ones_h,
          sums_o, cnt_o, deg_o,
          src_v, dst_v, rel_v, rows_a, rows_b, ones_v, sums_s, cnt_s, deg_s,
          sem_a, sem_b):
        cid = lax.axis_index("c")
        sid = lax.axis_index("s")
        wid = cid * NS + sid
        pltpu.sync_copy(ones_h, ones_v)
        pltpu.sync_copy(zd_h.at[pl.ds(sid * RPE, RPE)],
                        deg_s.at[pl.ds(sid * RPE, RPE)])
        pltpu.sync_copy(zs_h.at[pl.ds(sid * RPR, RPR)],
                        sums_s.at[pl.ds(sid * RPR, RPR)])
        pltpu.sync_copy(zc_h.at[pl.ds(sid * RPR, RPR)],
                        cnt_s.at[pl.ds(sid * RPR, RPR)])

        plsc.subcore_barrier()

        bufs = ((rows_a, sem_a), (rows_b, sem_b))

        def scatter(j, rv):
            pltpu.sync_copy(rv, sums_s.at[rel_v.at[j]], add=True)
            pltpu.sync_copy(ones_v, cnt_s.at[rel_v.at[j]], add=True)
            pltpu.sync_copy(ones_v, deg_s.at[dst_v.at[j]], add=True)

        def sblock(s, carry):
            pltpu.sync_copy(src_h.at[wid, pl.ds(s * SUP, SUP)], src_v)
            pltpu.sync_copy(dst_h.at[wid, pl.ds(s * SUP, SUP)], dst_v)
            pltpu.sync_copy(rel_h.at[wid, pl.ds(s * SUP, SUP)], rel_v)
            cps = []
            for j in range(SUP):
                rv, sa = bufs[j % 2]
                cps.append(pltpu.async_copy(ent_h.at[src_v.at[j]], rv, sa))
                if j >= 1:
                    cps[j - 1].wait()
                    scatter(j - 1, bufs[(j - 1) % 2][0])
            cps[SUP - 1].wait()
            scatter(SUP - 1, bufs[(SUP - 1) % 2][0])
            return carry

        lax.fori_loop(0, NSB, sblock, 0)
        plsc.subcore_barrier()

        pltpu.sync_copy(deg_s.at[pl.ds(sid * RPE, RPE)],
                        deg_o.at[cid, pl.ds(sid * RPE, RPE)])
        pltpu.sync_copy(sums_s.at[pl.ds(sid * RPR, RPR)],
                        sums_o.at[cid, pl.ds(sid * RPR, RPR)])
        pltpu.sync_copy(cnt_s.at[pl.ds(sid * RPR, RPR)],
                        cnt_o.at[cid, pl.ds(sid * RPR, RPR)])

    return k(src_r, dst_r, rel_r, ent_e, zsum, zcnt, zdeg, ones16)


def _edge_agg(src_r, dst_r, rel_r, hw, relw, zagg):
    mesh = plsc.VectorSubcoreMesh(**_MESH)

    @functools.partial(
        pl.kernel,
        out_type=jax.ShapeDtypeStruct((NC, NPE, H), jnp.float32),
        mesh=mesh,
        scratch_types=[
            pltpu.VMEM((SUP, CHUNK), jnp.int32),
            pltpu.VMEM((SUP, CHUNK), jnp.int32),
            pltpu.VMEM((SUP, CHUNK), jnp.int32),
            pltpu.VMEM((CHUNK, H), jnp.float32),
            pltpu.VMEM((CHUNK, H), jnp.float32),
            pltpu.VMEM((CHUNK, H), jnp.float32),
            pltpu.VMEM((CHUNK, H), jnp.float32),
            pltpu.VMEM_SHARED((NPE, H), jnp.float32),
            pltpu.SemaphoreType.DMA,
            pltpu.SemaphoreType.DMA,
            pltpu.SemaphoreType.DMA,
            pltpu.SemaphoreType.DMA,
        ],
    )
    def k(src_h, dst_h, rel_h, hw_h, relw_h, za_h, agg_o,
          src_v, dst_v, rel_v, rows_a, rrows_a, rows_b, rrows_b, agg_s,
          sem_a, sem2_a, sem_b, sem2_b):
        cid = lax.axis_index("c")
        sid = lax.axis_index("s")
        wid = cid * NS + sid
        pltpu.sync_copy(za_h.at[pl.ds(sid * RPE, RPE)],
                        agg_s.at[pl.ds(sid * RPE, RPE)])
        plsc.subcore_barrier()

        bufs = ((rows_a, rrows_a, sem_a, sem2_a),
                (rows_b, rrows_b, sem_b, sem2_b))

        def sblock(s, carry):
            pltpu.sync_copy(src_h.at[wid, pl.ds(s * SUP, SUP)], src_v)
            pltpu.sync_copy(dst_h.at[wid, pl.ds(s * SUP, SUP)], dst_v)
            pltpu.sync_copy(rel_h.at[wid, pl.ds(s * SUP, SUP)], rel_v)
            gat = []
            sca = {}
            for j in range(SUP):
                if j >= 2:
                    for c in sca.pop(j - 2):
                        c.wait()
                rv, rr, sa, sb = bufs[j % 2]
                gat.append((pltpu.async_copy(hw_h.at[src_v.at[j]], rv, sa),
                            pltpu.async_copy(relw_h.at[rel_v.at[j]], rr, sb)))
                if j >= 1:
                    jj = j - 1
                    pv, pr, psa, psb = bufs[jj % 2]
                    gat[jj][0].wait()
                    gat[jj][1].wait()
                    sca[jj] = (
                        pltpu.async_copy(pv, agg_s.at[dst_v.at[jj]], psa,
                                         add=True),
                        pltpu.async_copy(pr, agg_s.at[dst_v.at[jj]], psb,
                                         add=True))
            jj = SUP - 1
            pv, pr, psa, psb = bufs[jj % 2]
            gat[jj][0].wait()
            gat[jj][1].wait()
            sca[jj] = (
                pltpu.async_copy(pv, agg_s.at[dst_v.at[jj]], psa, add=True),
                pltpu.async_copy(pr, agg_s.at[dst_v.at[jj]], psb, add=True))
            for k in sorted(sca):
                for c in sca[k]:
                    c.wait()
            return carry

        lax.fori_loop(0, NSB, sblock, 0)
        plsc.subcore_barrier()
        pltpu.sync_copy(agg_s.at[pl.ds(sid * RPE, RPE)],
                        agg_o.at[cid, pl.ds(sid * RPE, RPE)])

    return k(src_r, dst_r, rel_r, hw, relw, zagg)


def _gather3(ent_f, n_rel, subj, obj, rel):
    mesh = plsc.VectorSubcoreMesh(**_MESH)

    @functools.partial(
        pl.kernel,
        out_type=[
            jax.ShapeDtypeStruct((B, H), jnp.float32),
            jax.ShapeDtypeStruct((B, H), jnp.float32),
            jax.ShapeDtypeStruct((B, H), jnp.float32),
        ],
        mesh=mesh,
        scratch_types=[
            pltpu.VMEM((BPT,), jnp.int32),
            pltpu.VMEM((BPT, H), jnp.float32),
            pltpu.SemaphoreType.DMA,
        ],
    )
    def k(entf_h, nrel_h, subj_h, obj_h, rel_h, e1_o, e2_o, rg_o,
          idx_v, rows_v, sem):
        cid = lax.axis_index("c")
        sid = lax.axis_index("s")
        wid = cid * NS + sid
        base = wid * BPT
        pltpu.sync_copy(subj_h.at[pl.ds(base, BPT)], idx_v)
        pltpu.async_copy(entf_h.at[idx_v], rows_v, sem).wait()
        pltpu.sync_copy(rows_v, e1_o.at[pl.ds(base, BPT)])
        pltpu.sync_copy(obj_h.at[pl.ds(base, BPT)], idx_v)
        pltpu.async_copy(entf_h.at[idx_v], rows_v, sem).wait()
        pltpu.sync_copy(rows_v, e2_o.at[pl.ds(base, BPT)])
        pltpu.sync_copy(rel_h.at[pl.ds(base, BPT)], idx_v)
        pltpu.async_copy(nrel_h.at[idx_v], rows_v, sem).wait()
        pltpu.sync_copy(rows_v, rg_o.at[pl.ds(base, BPT)])

    return k(ent_f, n_rel, subj, obj, rel)


def _edge_stats_xla(src_r, dst_r, rel_r, ent_e, zsum, zcnt, zdeg, ones16):
    sums, cnt, deg = [], [], []
    for c in range(NC):
        s = src_r[c * NS:(c + 1) * NS].reshape(-1)
        d = dst_r[c * NS:(c + 1) * NS].reshape(-1)
        r = rel_r[c * NS:(c + 1) * NS].reshape(-1)
        sums.append(jax.ops.segment_sum(ent_e[s], r, num_segments=NPR))
        o = jnp.ones((s.shape[0], H), jnp.float32)
        cnt.append(jax.ops.segment_sum(o, r, num_segments=NPR))
        deg.append(jax.ops.segment_sum(o, d, num_segments=NPE))
    return jnp.stack(sums), jnp.stack(cnt), jnp.stack(deg)


def _edge_agg_xla(src_r, dst_r, rel_r, hw, relw, zagg):
    out = []
    for c in range(NC):
        s = src_r[c * NS:(c + 1) * NS].reshape(-1)
        d = dst_r[c * NS:(c + 1) * NS].reshape(-1)
        r = rel_r[c * NS:(c + 1) * NS].reshape(-1)
        out.append(jax.ops.segment_sum(hw[s] + relw[r], d, num_segments=NPE))
    return jnp.stack(out)


def _gather3_xla(ent_f, n_rel, subj, obj, rel):
    return ent_f[subj], ent_f[obj], n_rel[rel]


def _decoder_xla(e1, e2, convw_r, convb, fcw3t, fcb, table, n_out):
    zc = jnp.zeros((B, 1), jnp.float32)
    xs = []
    for x in (e1, e2):
        xs.append([jnp.concatenate([zc, x[:, :H - 1]], axis=1), x,
                   jnp.concatenate([x[:, 1:], zc], axis=1)])
    y = (xs[0][0][:, None, :] * convw_r[0, 0][None, :, None]
         + xs[0][1][:, None, :] * convw_r[0, 1][None, :, None]
         + xs[0][2][:, None, :] * convw_r[0, 2][None, :, None]
         + xs[1][0][:, None, :] * convw_r[1, 0][None, :, None]
         + xs[1][1][:, None, :] * convw_r[1, 1][None, :, None]
         + xs[1][2][:, None, :] * convw_r[1, 2][None, :, None]
         + convb[None, :, None])
    y = jnp.maximum(y, 0.0)
    t = jnp.einsum('bcl,clj->bj', y, fcw3t)
    t = jnp.maximum(t + fcb, 0.0)
    return t @ table.T


_RB = 2000
_NRB = N_ENT // _RB


def _full(shape):
    nd = len(shape)
    return pl.BlockSpec(shape, lambda i: (0,) * nd)


def _rows(w):
    return pl.BlockSpec((_RB, w), lambda i: (i, 0))


def _tc_l2norm(x):
    def body(x_ref, o_ref):
        o_ref[...] = _l2n(x_ref[...])

    return pl.pallas_call(
        body,
        grid=(_NRB,),
        in_specs=[_rows(H)],
        out_specs=_rows(H),
        out_shape=jax.ShapeDtypeStruct((N_ENT, H), jnp.float32),
    )(x)


def _tc_relgru(rel_embeds, sums_p, cnt_p, w_ih, w_hh, b_ih, b_hh, wr1, wr2):
    def body(re_ref, sums_ref, cnt_ref, wih_ref, whh_ref, bih_ref, bhh_ref,
             wr1_ref, wr2_ref, nrel_ref, rw1_ref, rw2_ref):
        rel_emb = re_ref[...]
        rel_e = _l2n(rel_emb)
        sums = sums_ref[0] + sums_ref[1]
        cnts = cnt_ref[0, :, 0] + cnt_ref[1, :, 0]
        rel_ent = jnp.where(cnts[:, None] > 0,
                            sums / jnp.maximum(cnts, 1.0)[:, None], 0.0)
        r_rel = jnp.concatenate([rel_emb, rel_ent], axis=-1)
        dn = (((1,), (1,)), ((), ()))
        gi = lax.dot_general(r_rel, wih_ref[...], dn,
                             preferred_element_type=jnp.float32) + bih_ref[...]
        gh = lax.dot_general(rel_e, whh_ref[...], dn,
                             preferred_element_type=jnp.float32) + bhh_ref[...]
        i_r, i_z, i_n = gi[:, :H], gi[:, H:2 * H], gi[:, 2 * H:]
        h_r, h_z, h_n = gh[:, :H], gh[:, H:2 * H], gh[:, 2 * H:]
        r = jax.nn.sigmoid(i_r + h_r)
        z = jax.nn.sigmoid(i_z + h_z)
        n = jnp.tanh(i_n + r * h_n)
        n_rel = _l2n((1.0 - z) * n + z * rel_e)
        nrel_ref[...] = n_rel
        rw1_ref[...] = lax.dot_general(n_rel, wr1_ref[...], dn,
                                       preferred_element_type=jnp.float32)
        rw2_ref[...] = lax.dot_general(n_rel, wr2_ref[...], dn,
                                       preferred_element_type=jnp.float32)

    return pl.pallas_call(
        body,
        grid=(1,),
        in_specs=[_full((N_REL, H)), _full((NC, N_REL, H)),
                  _full((NC, N_REL, H)), _full((3 * H, 2 * H)),
                  _full((3 * H, H)), _full((3 * H,)), _full((3 * H,)),
                  _full((H, H)), _full((H, H))],
        out_specs=[_full((N_REL, H))] * 3,
        out_shape=[jax.ShapeDtypeStruct((N_REL, H), jnp.float32)] * 3,
    )(rel_embeds, sums_p, cnt_p, w_ih, w_hh, b_ih, b_hh, wr1, wr2)


def _tc_entprep(ent_e, deg0, deg1, wr1, wsl1, wel1, lin_w, lin_b):
    def body(e_ref, d0_ref, d1_ref, wr_ref, wsl_ref, wel_ref, lw_ref, lb_ref,
             hw_ref, s_ref, u_ref):
        e = e_ref[...]
        deg = d0_ref[:, :1] + d1_ref[:, :1]
        dn = (((1,), (1,)), ((), ()))
        hw_ref[...] = lax.dot_general(e, wr_ref[...], dn,
                                      preferred_element_type=jnp.float32)
        msl = lax.dot_general(e, wsl_ref[...], dn,
                              preferred_element_type=jnp.float32)
        mel = lax.dot_general(e, wel_ref[...], dn,
                              preferred_element_type=jnp.float32)
        s_ref[...] = jnp.where(deg == 0.0, mel, msl)
        u_ref[...] = jax.nn.sigmoid(
            lax.dot_general(e, lw_ref[...], dn,
                            preferred_element_type=jnp.float32) + lb_ref[...])

    return pl.pallas_call(
        body,
        grid=(_NRB,),
        in_specs=[_rows(H), _rows(H), _rows(H), _full((H, H)),
                  _full((H, H)), _full((H, H)), _full((H, H)), _full((H,))],
        out_specs=[_rows(H)] * 3,
        out_shape=[jax.ShapeDtypeStruct((N_ENT, H), jnp.float32)] * 3,
    )(ent_e, deg0, deg1, wr1, wsl1, wel1, lin_w, lin_b)


def _tc_layer2prep(agg0, agg1, self1, deg0, deg1, wr2, wsl2, wel2):
    def body(a0_ref, a1_ref, s1_ref, d0_ref, d1_ref, wr_ref, wsl_ref, wel_ref,
             hw_ref, s2_ref):
        deg = d0_ref[:, :1] + d1_ref[:, :1]
        norm = 1.0 / jnp.maximum(deg, 1.0)
        x = (a0_ref[...] + a1_ref[...]) * norm + s1_ref[...]
        h1 = _lrelu(x)
        dn = (((1,), (1,)), ((), ()))
        hw_ref[...] = lax.dot_general(h1, wr_ref[...], dn,
                                      preferred_element_type=jnp.float32)
        msl = lax.dot_general(h1, wsl_ref[...], dn,
                              preferred_element_type=jnp.float32)
        mel = lax.dot_general(h1, wel_ref[...], dn,
                              preferred_element_type=jnp.float32)
        s2_ref[...] = jnp.where(deg == 0.0, mel, msl)

    return pl.pallas_call(
        body,
        grid=(_NRB,),
        in_specs=[_rows(H), _rows(H), _rows(H), _rows(H), _rows(H),
                  _full((H, H)), _full((H, H)), _full((H, H))],
        out_specs=[_rows(H)] * 2,
        out_shape=[jax.ShapeDtypeStruct((N_ENT, H), jnp.float32)] * 2,
    )(agg0, agg1, self1, deg0, deg1, wr2, wsl2, wel2)


def _tc_entfinal(agg0, agg1, self2, deg0, deg1, ent_e, u):
    def body(a0_ref, a1_ref, s2_ref, d0_ref, d1_ref, e_ref, u_ref, o_ref):
        deg = d0_ref[:, :1] + d1_ref[:, :1]
        norm = 1.0 / jnp.maximum(deg, 1.0)
        x = (a0_ref[...] + a1_ref[...]) * norm + s2_ref[...]
        w_ent = _l2n(_lrelu(x))
        e = e_ref[...]
        o_ref[...] = e + u_ref[...] * (w_ent - e)

    return pl.pallas_call(
        body,
        grid=(_NRB,),
        in_specs=[_rows(H)] * 3 + [_rows(H), _rows(H), _rows(H), _rows(H)],
        out_specs=_rows(H),
        out_shape=jax.ShapeDtypeStruct((N_ENT, H), jnp.float32),
    )(agg0, agg1, self2, deg0, deg1, ent_e, u)


_BB = 128


def _tc_dec_feat(e1, e2, convw_r, convb, fcw_t, fcb):
    def body(e1_ref, e2_ref, cw_ref, cb_ref, fw_ref, fb_ref, t_ref):
        x1 = e1_ref[...]
        x2 = e2_ref[...]
        z = jnp.zeros((_BB, 1), jnp.float32)
        x1m = jnp.concatenate([z, x1[:, :H - 1]], axis=1)
        x1p = jnp.concatenate([x1[:, 1:], z], axis=1)
        x2m = jnp.concatenate([z, x2[:, :H - 1]], axis=1)
        x2p = jnp.concatenate([x2[:, 1:], z], axis=1)
        cw = cw_ref[...]
        cb = cb_ref[...]
        y = (x1m[:, None, :] * cw[0, 0][None, :, None]
             + x1[:, None, :] * cw[0, 1][None, :, None]
             + x1p[:, None, :] * cw[0, 2][None, :, None]
             + x2m[:, None, :] * cw[1, 0][None, :, None]
             + x2[:, None, :] * cw[1, 1][None, :, None]
             + x2p[:, None, :] * cw[1, 2][None, :, None]
             + cb[None, :, None])
        y = jnp.maximum(y, 0.0).reshape(_BB, CH * H)
        t_ref[...] = jnp.maximum(
            jnp.dot(y, fw_ref[...], preferred_element_type=jnp.float32)
            + fb_ref[...], 0.0)

    return pl.pallas_call(
        body,
        grid=(B // _BB,),
        in_specs=[pl.BlockSpec((_BB, H), lambda i: (i, 0)),
                  pl.BlockSpec((_BB, H), lambda i: (i, 0)),
                  _full((2, K, CH)), _full((CH,)), _full((CH * H, H)),
                  _full((H,))],
        out_specs=pl.BlockSpec((_BB, H), lambda i: (i, 0)),
        out_shape=jax.ShapeDtypeStruct((B, H), jnp.float32),
    )(e1, e2, convw_r, convb, fcw_t, fcb)


def _tc_logits(t, table, n_pad, vb):
    def body(t_ref, tab_ref, o_ref):
        dn = (((1,), (1,)), ((), ()))
        o_ref[...] = lax.dot_general(t_ref[...], tab_ref[...], dn,
                                     preferred_element_type=jnp.float32)

    return pl.pallas_call(
        body,
        grid=(B // _BB, n_pad // vb),
        in_specs=[pl.BlockSpec((_BB, H), lambda i, j: (i, 0)),
                  pl.BlockSpec((vb, H), lambda i, j: (j, 0))],
        out_specs=pl.BlockSpec((_BB, vb), lambda i, j: (i, j)),
        out_shape=jax.ShapeDtypeStruct((B, n_pad), jnp.float32),
    )(t, table)


@jax.jit
def kernel(ent_embeds, rel_embeds, W_r1, W_sl1, W_el1, W_r2, W_sl2, W_el2,
           lin_w, lin_b, gru_w_ih, gru_w_hh, gru_b_ih, gru_b_hh, convR_w,
           convR_b, fcR_w, fcR_b, convE_w, convE_b, fcE_w, fcE_b, edge_index,
           edge_rel, subj, rel, obj):
    npad = E_PAD - E
    src_r = jnp.concatenate(
        [edge_index[0].astype(jnp.int32), jnp.zeros((npad,), jnp.int32)]
    ).reshape(NW, NCHUNK, CHUNK)
    dst_r = jnp.concatenate(
        [edge_index[1].astype(jnp.int32),
         jnp.full((npad,), N_ENT, jnp.int32)]
    ).reshape(NW, NCHUNK, CHUNK)
    rel_r = jnp.concatenate(
        [edge_rel.astype(jnp.int32), jnp.full((npad,), N_REL, jnp.int32)]
    ).reshape(NW, NCHUNK, CHUNK)
    subj_i = subj.astype(jnp.int32)
    obj_i = obj.astype(jnp.int32)
    rel_i = rel.astype(jnp.int32)

    zsum = jnp.zeros((NPR, H), jnp.float32)
    zcnt = jnp.zeros((NPR, H), jnp.float32)
    zdeg = jnp.zeros((NPE, H), jnp.float32)
    zagg = jnp.zeros((NPE, H), jnp.float32)
    ones16 = jnp.ones((CHUNK, H), jnp.float32)

    ent_e = _tc_l2norm(ent_embeds)

    sums_pp, cnt_pp, deg_pp = _edge_stats(src_r, dst_r, rel_r, ent_e,
                                          zsum, zcnt, zdeg, ones16)
    sums_p = sums_pp[:, :N_REL]
    cnt_p = cnt_pp[:, :N_REL]
    deg0 = deg_pp[0, :N_ENT]
    deg1 = deg_pp[1, :N_ENT]

    n_rel, relW1, relW2 = _tc_relgru(rel_embeds, sums_p, cnt_p, gru_w_ih,
                                     gru_w_hh, gru_b_ih, gru_b_hh, W_r1, W_r2)
    zrel = jnp.zeros((NPR - N_REL, H), jnp.float32)
    relW1 = jnp.concatenate([relW1, zrel])
    relW2 = jnp.concatenate([relW2, zrel])

    hW1, self1, u = _tc_entprep(ent_e, deg0, deg1, W_r1, W_sl1, W_el1,
                                lin_w, lin_b)

    agg1_p = _edge_agg(src_r, dst_r, rel_r, hW1, relW1, zagg)
    hW2, self2 = _tc_layer2prep(agg1_p[0, :N_ENT], agg1_p[1, :N_ENT], self1,
                                deg0, deg1, W_r2, W_sl2, W_el2)

    agg2_p = _edge_agg(src_r, dst_r, rel_r, hW2, relW2, zagg)
    ent_f = _tc_entfinal(agg2_p[0, :N_ENT], agg2_p[1, :N_ENT], self2, deg0,
                         deg1, ent_e, u)

    e1, e2, rg = _gather3(ent_f, n_rel, subj_i, obj_i, rel_i)

    convR_r = convR_w.transpose(1, 2, 0)
    convE_r = convE_w.transpose(1, 2, 0)

    tR = _tc_dec_feat(e1, e2, convR_r, convR_b, fcR_w.T, fcR_b)
    tE = _tc_dec_feat(e1, rg, convE_r, convE_b, fcE_w.T, fcE_b)

    nrel_pad = jnp.concatenate(
        [n_rel, jnp.zeros((NPR - N_REL, H), jnp.float32)])
    entf_pad = jnp.concatenate(
        [ent_f, jnp.zeros((NPE - N_ENT, H), jnp.float32)])
    rel_logit = _tc_logits(tR, nrel_pad, NPR, NPR)[:, :N_REL]
    obj_logit = _tc_logits(tE, entf_pad, NPE, 1280)[:, :N_ENT]
    return (obj_logit, rel_logit)


# speedup vs baseline: 2.6270x; 1.0048x over previous
import functools

import jax
import jax.numpy as jnp
from jax import lax
from jax.experimental import pallas as pl
from jax.experimental.pallas import tpu as pltpu
from jax.experimental.pallas import tpu_sc as plsc

N_ENT = 10000
N_REL = 200
H = 128
CH = 50
K = 3
B = 1024
E = 320000
RRELU_SLOPE = (1.0 / 8.0 + 1.0 / 3.0) / 2.0

NC = 2
NS = 16
NPE = 10240
NPR = 256
RPE = NPE // NS
RPR = NPR // NS
NW = NC * NS
CHUNK = 80
NCHUNK = 128
SUP = 8
NSB = NCHUNK // SUP
EPT = NCHUNK * CHUNK
E_PAD = NW * EPT
BPT = B // NW

_MESH = dict(core_axis_name="c", subcore_axis_name="s", num_cores=NC,
             num_subcores=NS)


def _l2n(x):
    n = jnp.sqrt(jnp.sum(x * x, axis=-1, keepdims=True))
    return x / jnp.maximum(n, 1e-12)


def _lrelu(x):
    return jnp.where(x >= 0, x, RRELU_SLOPE * x)


def _edge_stats(src_r, dst_r, rel_r, ent_e, zsum, zcnt, zdeg, ones16):
    mesh = plsc.VectorSubcoreMesh(**_MESH)

    @functools.partial(
        pl.kernel,
        out_type=[
            jax.ShapeDtypeStruct((NC, NPR, H), jnp.float32),
            jax.ShapeDtypeStruct((NC, NPR, H), jnp.float32),
            jax.ShapeDtypeStruct((NC, NPE, H), jnp.float32),
        ],
        mesh=mesh,
        scratch_types=[
            pltpu.VMEM((SUP, CHUNK), jnp.int32),
            pltpu.VMEM((SUP, CHUNK), jnp.int32),
            pltpu.VMEM((SUP, CHUNK), jnp.int32),
            pltpu.VMEM((CHUNK, H), jnp.float32),
            pltpu.VMEM((CHUNK, H), jnp.float32),
            pltpu.VMEM((CHUNK, H), jnp.float32),
            pltpu.VMEM_SHARED((NPR, H), jnp.float32),
            pltpu.VMEM_SHARED((NPR, H), jnp.float32),
            pltpu.VMEM_SHARED((NPE, H), jnp.float32),
            pltpu.SemaphoreType.DMA,
            pltpu.SemaphoreType.DMA,
            pltpu.SemaphoreType.DMA,
            pltpu.SemaphoreType.DMA,
            pltpu.SemaphoreType.DMA,
        ],
    )
    def k(src_h, dst_h, rel_h, ent_h, zs_h, zc_h, zd_h, ones_h,
          sums_o, cnt_o, deg_o,
          src_v, dst_v, rel_v, rows_a, rows_b, ones_v, sums_s, cnt_s, deg_s,
          sem_a, sem_b, ssem_a, ssem_b, osem):
        cid = lax.axis_index("c")
        sid = lax.axis_index("s")
        wid = cid * NS + sid
        pltpu.sync_copy(ones_h, ones_v)
        pltpu.sync_copy(zd_h.at[pl.ds(sid * RPE, RPE)],
                        deg_s.at[pl.ds(sid * RPE, RPE)])
        pltpu.sync_copy(zs_h.at[pl.ds(sid * RPR, RPR)],
                        sums_s.at[pl.ds(sid * RPR, RPR)])
        pltpu.sync_copy(zc_h.at[pl.ds(sid * RPR, RPR)],
                        cnt_s.at[pl.ds(sid * RPR, RPR)])

        plsc.subcore_barrier()

        bufs = ((rows_a, sem_a, ssem_a), (rows_b, sem_b, ssem_b))

        def sblock(s, carry):
            pltpu.sync_copy(src_h.at[wid, pl.ds(s * SUP, SUP)], src_v)
            pltpu.sync_copy(dst_h.at[wid, pl.ds(s * SUP, SUP)], dst_v)
            pltpu.sync_copy(rel_h.at[wid, pl.ds(s * SUP, SUP)], rel_v)
            gat = []
            sca = {}
            ones_cps = []

            def scatter(j):
                rv, _, ss = bufs[j % 2]
                sca[j] = pltpu.async_copy(rv, sums_s.at[rel_v.at[j]], ss,
                                          add=True)
                ones_cps.append(
                    pltpu.async_copy(ones_v, cnt_s.at[rel_v.at[j]], osem,
                                     add=True))
                ones_cps.append(
                    pltpu.async_copy(ones_v, deg_s.at[dst_v.at[j]], osem,
                                     add=True))

            for j in range(SUP):
                if j >= 2:
                    sca.pop(j - 2).wait()
                rv, sa, _ = bufs[j % 2]
                gat.append(pltpu.async_copy(ent_h.at[src_v.at[j]], rv, sa))
                if j >= 1:
                    gat[j - 1].wait()
                    scatter(j - 1)
            gat[SUP - 1].wait()
            scatter(SUP - 1)
            sca.pop(SUP - 2).wait()
            sca.pop(SUP - 1).wait()
            for c in ones_cps:
                c.wait()
            return carry

        lax.fori_loop(0, NSB, sblock, 0)
        plsc.subcore_barrier()

        pltpu.sync_copy(deg_s.at[pl.ds(sid * RPE, RPE)],
                        deg_o.at[cid, pl.ds(sid * RPE, RPE)])
        pltpu.sync_copy(sums_s.at[pl.ds(sid * RPR, RPR)],
                        sums_o.at[cid, pl.ds(sid * RPR, RPR)])
        pltpu.sync_copy(cnt_s.at[pl.ds(sid * RPR, RPR)],
                        cnt_o.at[cid, pl.ds(sid * RPR, RPR)])

    return k(src_r, dst_r, rel_r, ent_e, zsum, zcnt, zdeg, ones16)


def _edge_agg(src_r, dst_r, rel_r, hw, relw, zagg):
    mesh = plsc.VectorSubcoreMesh(**_MESH)

    @functools.partial(
        pl.kernel,
        out_type=jax.ShapeDtypeStruct((NC, NPE, H), jnp.float32),
        mesh=mesh,
        scratch_types=[
            pltpu.VMEM((SUP, CHUNK), jnp.int32),
            pltpu.VMEM((SUP, CHUNK), jnp.int32),
            pltpu.VMEM((SUP, CHUNK), jnp.int32),
            pltpu.VMEM((CHUNK, H), jnp.float32),
            pltpu.VMEM((CHUNK, H), jnp.float32),
            pltpu.VMEM((CHUNK, H), jnp.float32),
            pltpu.VMEM((CHUNK, H), jnp.float32),
            pltpu.VMEM_SHARED((NPE, H), jnp.float32),
            pltpu.SemaphoreType.DMA,
            pltpu.SemaphoreType.DMA,
            pltpu.SemaphoreType.DMA,
            pltpu.SemaphoreType.DMA,
        ],
    )
    def k(src_h, dst_h, rel_h, hw_h, relw_h, za_h, agg_o,
          src_v, dst_v, rel_v, rows_a, rrows_a, rows_b, rrows_b, agg_s,
          sem_a, sem2_a, sem_b, sem2_b):
        cid = lax.axis_index("c")
        sid = lax.axis_index("s")
        wid = cid * NS + sid
        pltpu.sync_copy(za_h.at[pl.ds(sid * RPE, RPE)],
                        agg_s.at[pl.ds(sid * RPE, RPE)])
        plsc.subcore_barrier()

        bufs = ((rows_a, rrows_a, sem_a, sem2_a),
                (rows_b, rrows_b, sem_b, sem2_b))

        def sblock(s, carry):
            pltpu.sync_copy(src_h.at[wid, pl.ds(s * SUP, SUP)], src_v)
            pltpu.sync_copy(dst_h.at[wid, pl.ds(s * SUP, SUP)], dst_v)
            pltpu.sync_copy(rel_h.at[wid, pl.ds(s * SUP, SUP)], rel_v)
            gat = []
            sca = {}
            for j in range(SUP):
                if j >= 2:
                    for c in sca.pop(j - 2):
                        c.wait()
                rv, rr, sa, sb = bufs[j % 2]
                gat.append((pltpu.async_copy(hw_h.at[src_v.at[j]], rv, sa),
                            pltpu.async_copy(relw_h.at[rel_v.at[j]], rr, sb)))
                if j >= 1:
                    jj = j - 1
                    pv, pr, psa, psb = bufs[jj % 2]
                    gat[jj][0].wait()
                    gat[jj][1].wait()
                    sca[jj] = (
                        pltpu.async_copy(pv, agg_s.at[dst_v.at[jj]], psa,
                                         add=True),
                        pltpu.async_copy(pr, agg_s.at[dst_v.at[jj]], psb,
                                         add=True))
            jj = SUP - 1
            pv, pr, psa, psb = bufs[jj % 2]
            gat[jj][0].wait()
            gat[jj][1].wait()
            sca[jj] = (
                pltpu.async_copy(pv, agg_s.at[dst_v.at[jj]], psa, add=True),
                pltpu.async_copy(pr, agg_s.at[dst_v.at[jj]], psb, add=True))
            for k in sorted(sca):
                for c in sca[k]:
                    c.wait()
            return carry

        lax.fori_loop(0, NSB, sblock, 0)
        plsc.subcore_barrier()
        pltpu.sync_copy(agg_s.at[pl.ds(sid * RPE, RPE)],
                        agg_o.at[cid, pl.ds(sid * RPE, RPE)])

    return k(src_r, dst_r, rel_r, hw, relw, zagg)


def _gather3(ent_f, n_rel, subj, obj, rel):
    mesh = plsc.VectorSubcoreMesh(**_MESH)

    @functools.partial(
        pl.kernel,
        out_type=[
            jax.ShapeDtypeStruct((B, H), jnp.float32),
            jax.ShapeDtypeStruct((B, H), jnp.float32),
            jax.ShapeDtypeStruct((B, H), jnp.float32),
        ],
        mesh=mesh,
        scratch_types=[
            pltpu.VMEM((BPT,), jnp.int32),
            pltpu.VMEM((BPT, H), jnp.float32),
            pltpu.SemaphoreType.DMA,
        ],
    )
    def k(entf_h, nrel_h, subj_h, obj_h, rel_h, e1_o, e2_o, rg_o,
          idx_v, rows_v, sem):
        cid = lax.axis_index("c")
        sid = lax.axis_index("s")
        wid = cid * NS + sid
        base = wid * BPT
        pltpu.sync_copy(subj_h.at[pl.ds(base, BPT)], idx_v)
        pltpu.async_copy(entf_h.at[idx_v], rows_v, sem).wait()
        pltpu.sync_copy(rows_v, e1_o.at[pl.ds(base, BPT)])
        pltpu.sync_copy(obj_h.at[pl.ds(base, BPT)], idx_v)
        pltpu.async_copy(entf_h.at[idx_v], rows_v, sem).wait()
        pltpu.sync_copy(rows_v, e2_o.at[pl.ds(base, BPT)])
        pltpu.sync_copy(rel_h.at[pl.ds(base, BPT)], idx_v)
        pltpu.async_copy(nrel_h.at[idx_v], rows_v, sem).wait()
        pltpu.sync_copy(rows_v, rg_o.at[pl.ds(base, BPT)])

    return k(ent_f, n_rel, subj, obj, rel)


def _edge_stats_xla(src_r, dst_r, rel_r, ent_e, zsum, zcnt, zdeg, ones16):
    sums, cnt, deg = [], [], []
    for c in range(NC):
        s = src_r[c * NS:(c + 1) * NS].reshape(-1)
        d = dst_r[c * NS:(c + 1) * NS].reshape(-1)
        r = rel_r[c * NS:(c + 1) * NS].reshape(-1)
        sums.append(jax.ops.segment_sum(ent_e[s], r, num_segments=NPR))
        o = jnp.ones((s.shape[0], H), jnp.float32)
        cnt.append(jax.ops.segment_sum(o, r, num_segments=NPR))
        deg.append(jax.ops.segment_sum(o, d, num_segments=NPE))
    return jnp.stack(sums), jnp.stack(cnt), jnp.stack(deg)


def _edge_agg_xla(src_r, dst_r, rel_r, hw, relw, zagg):
    out = []
    for c in range(NC):
        s = src_r[c * NS:(c + 1) * NS].reshape(-1)
        d = dst_r[c * NS:(c + 1) * NS].reshape(-1)
        r = rel_r[c * NS:(c + 1) * NS].reshape(-1)
        out.append(jax.ops.segment_sum(hw[s] + relw[r], d, num_segments=NPE))
    return jnp.stack(out)


def _gather3_xla(ent_f, n_rel, subj, obj, rel):
    return ent_f[subj], ent_f[obj], n_rel[rel]


def _decoder_xla(e1, e2, convw_r, convb, fcw3t, fcb, table, n_out):
    zc = jnp.zeros((B, 1), jnp.float32)
    xs = []
    for x in (e1, e2):
        xs.append([jnp.concatenate([zc, x[:, :H - 1]], axis=1), x,
                   jnp.concatenate([x[:, 1:], zc], axis=1)])
    y = (xs[0][0][:, None, :] * convw_r[0, 0][None, :, None]
         + xs[0][1][:, None, :] * convw_r[0, 1][None, :, None]
         + xs[0][2][:, None, :] * convw_r[0, 2][None, :, None]
         + xs[1][0][:, None, :] * convw_r[1, 0][None, :, None]
         + xs[1][1][:, None, :] * convw_r[1, 1][None, :, None]
         + xs[1][2][:, None, :] * convw_r[1, 2][None, :, None]
         + convb[None, :, None])
    y = jnp.maximum(y, 0.0)
    t = jnp.einsum('bcl,clj->bj', y, fcw3t)
    t = jnp.maximum(t + fcb, 0.0)
    return t @ table.T


_RB = 2000
_NRB = N_ENT // _RB


def _full(shape):
    nd = len(shape)
    return pl.BlockSpec(shape, lambda i: (0,) * nd)


def _rows(w):
    return pl.BlockSpec((_RB, w), lambda i: (i, 0))


def _tc_l2norm(x):
    def body(x_ref, o_ref):
        o_ref[...] = _l2n(x_ref[...])

    return pl.pallas_call(
        body,
        grid=(_NRB,),
        in_specs=[_rows(H)],
        out_specs=_rows(H),
        out_shape=jax.ShapeDtypeStruct((N_ENT, H), jnp.float32),
    )(x)


def _tc_relgru(rel_embeds, sums_p, cnt_p, w_ih, w_hh, b_ih, b_hh, wr1, wr2):
    def body(re_ref, sums_ref, cnt_ref, wih_ref, whh_ref, bih_ref, bhh_ref,
             wr1_ref, wr2_ref, nrel_ref, rw1_ref, rw2_ref):
        rel_emb = re_ref[...]
        rel_e = _l2n(rel_emb)
        sums = sums_ref[0] + sums_ref[1]
        cnts = cnt_ref[0, :, 0] + cnt_ref[1, :, 0]
        rel_ent = jnp.where(cnts[:, None] > 0,
                            sums / jnp.maximum(cnts, 1.0)[:, None], 0.0)
        r_rel = jnp.concatenate([rel_emb, rel_ent], axis=-1)
        dn = (((1,), (1,)), ((), ()))
        gi = lax.dot_general(r_rel, wih_ref[...], dn,
                             preferred_element_type=jnp.float32) + bih_ref[...]
        gh = lax.dot_general(rel_e, whh_ref[...], dn,
                             preferred_element_type=jnp.float32) + bhh_ref[...]
        i_r, i_z, i_n = gi[:, :H], gi[:, H:2 * H], gi[:, 2 * H:]
        h_r, h_z, h_n = gh[:, :H], gh[:, H:2 * H], gh[:, 2 * H:]
        r = jax.nn.sigmoid(i_r + h_r)
        z = jax.nn.sigmoid(i_z + h_z)
        n = jnp.tanh(i_n + r * h_n)
        n_rel = _l2n((1.0 - z) * n + z * rel_e)
        nrel_ref[...] = n_rel
        rw1_ref[...] = lax.dot_general(n_rel, wr1_ref[...], dn,
                                       preferred_element_type=jnp.float32)
        rw2_ref[...] = lax.dot_general(n_rel, wr2_ref[...], dn,
                                       preferred_element_type=jnp.float32)

    return pl.pallas_call(
        body,
        grid=(1,),
        in_specs=[_full((N_REL, H)), _full((NC, N_REL, H)),
                  _full((NC, N_REL, H)), _full((3 * H, 2 * H)),
                  _full((3 * H, H)), _full((3 * H,)), _full((3 * H,)),
                  _full((H, H)), _full((H, H))],
        out_specs=[_full((N_REL, H))] * 3,
        out_shape=[jax.ShapeDtypeStruct((N_REL, H), jnp.float32)] * 3,
    )(rel_embeds, sums_p, cnt_p, w_ih, w_hh, b_ih, b_hh, wr1, wr2)


def _tc_entprep(ent_e, deg0, deg1, wr1, wsl1, wel1, lin_w, lin_b):
    def body(e_ref, d0_ref, d1_ref, wr_ref, wsl_ref, wel_ref, lw_ref, lb_ref,
             hw_ref, s_ref, u_ref):
        e = e_ref[...]
        deg = d0_ref[:, :1] + d1_ref[:, :1]
        dn = (((1,), (1,)), ((), ()))
        hw_ref[...] = lax.dot_general(e, wr_ref[...], dn,
                                      preferred_element_type=jnp.float32)
        msl = lax.dot_general(e, wsl_ref[...], dn,
                              preferred_element_type=jnp.float32)
        mel = lax.dot_general(e, wel_ref[...], dn,
                              preferred_element_type=jnp.float32)
        s_ref[...] = jnp.where(deg == 0.0, mel, msl)
        u_ref[...] = jax.nn.sigmoid(
            lax.dot_general(e, lw_ref[...], dn,
                            preferred_element_type=jnp.float32) + lb_ref[...])

    return pl.pallas_call(
        body,
        grid=(_NRB,),
        in_specs=[_rows(H), _rows(H), _rows(H), _full((H, H)),
                  _full((H, H)), _full((H, H)), _full((H, H)), _full((H,))],
        out_specs=[_rows(H)] * 3,
        out_shape=[jax.ShapeDtypeStruct((N_ENT, H), jnp.float32)] * 3,
    )(ent_e, deg0, deg1, wr1, wsl1, wel1, lin_w, lin_b)


def _tc_layer2prep(agg0, agg1, self1, deg0, deg1, wr2, wsl2, wel2):
    def body(a0_ref, a1_ref, s1_ref, d0_ref, d1_ref, wr_ref, wsl_ref, wel_ref,
             hw_ref, s2_ref):
        deg = d0_ref[:, :1] + d1_ref[:, :1]
        norm = 1.0 / jnp.maximum(deg, 1.0)
        x = (a0_ref[...] + a1_ref[...]) * norm + s1_ref[...]
        h1 = _lrelu(x)
        dn = (((1,), (1,)), ((), ()))
        hw_ref[...] = lax.dot_general(h1, wr_ref[...], dn,
                                      preferred_element_type=jnp.float32)
        msl = lax.dot_general(h1, wsl_ref[...], dn,
                              preferred_element_type=jnp.float32)
        mel = lax.dot_general(h1, wel_ref[...], dn,
                              preferred_element_type=jnp.float32)
        s2_ref[...] = jnp.where(deg == 0.0, mel, msl)

    return pl.pallas_call(
        body,
        grid=(_NRB,),
        in_specs=[_rows(H), _rows(H), _rows(H), _rows(H), _rows(H),
                  _full((H, H)), _full((H, H)), _full((H, H))],
        out_specs=[_rows(H)] * 2,
        out_shape=[jax.ShapeDtypeStruct((N_ENT, H), jnp.float32)] * 2,
    )(agg0, agg1, self1, deg0, deg1, wr2, wsl2, wel2)


def _tc_entfinal(agg0, agg1, self2, deg0, deg1, ent_e, u):
    def body(a0_ref, a1_ref, s2_ref, d0_ref, d1_ref, e_ref, u_ref, o_ref):
        deg = d0_ref[:, :1] + d1_ref[:, :1]
        norm = 1.0 / jnp.maximum(deg, 1.0)
        x = (a0_ref[...] + a1_ref[...]) * norm + s2_ref[...]
        w_ent = _l2n(_lrelu(x))
        e = e_ref[...]
        o_ref[...] = e + u_ref[...] * (w_ent - e)

    return pl.pallas_call(
        body,
        grid=(_NRB,),
        in_specs=[_rows(H)] * 3 + [_rows(H), _rows(H), _rows(H), _rows(H)],
        out_specs=_rows(H),
        out_shape=jax.ShapeDtypeStruct((N_ENT, H), jnp.float32),
    )(agg0, agg1, self2, deg0, deg1, ent_e, u)


_BB = 128


def _tc_dec_feat(e1, e2, convw_r, convb, fcw_t, fcb):
    def body(e1_ref, e2_ref, cw_ref, cb_ref, fw_ref, fb_ref, t_ref):
        x1 = e1_ref[...]
        x2 = e2_ref[...]
        z = jnp.zeros((_BB, 1), jnp.float32)
        x1m = jnp.concatenate([z, x1[:, :H - 1]], axis=1)
        x1p = jnp.concatenate([x1[:, 1:], z], axis=1)
        x2m = jnp.concatenate([z, x2[:, :H - 1]], axis=1)
        x2p = jnp.concatenate([x2[:, 1:], z], axis=1)
        cw = cw_ref[...]
        cb = cb_ref[...]
        y = (x1m[:, None, :] * cw[0, 0][None, :, None]
             + x1[:, None, :] * cw[0, 1][None, :, None]
             + x1p[:, None, :] * cw[0, 2][None, :, None]
             + x2m[:, None, :] * cw[1, 0][None, :, None]
             + x2[:, None, :] * cw[1, 1][None, :, None]
             + x2p[:, None, :] * cw[1, 2][None, :, None]
             + cb[None, :, None])
        y = jnp.maximum(y, 0.0).reshape(_BB, CH * H)
        t_ref[...] = jnp.maximum(
            jnp.dot(y, fw_ref[...], preferred_element_type=jnp.float32)
            + fb_ref[...], 0.0)

    return pl.pallas_call(
        body,
        grid=(B // _BB,),
        in_specs=[pl.BlockSpec((_BB, H), lambda i: (i, 0)),
                  pl.BlockSpec((_BB, H), lambda i: (i, 0)),
                  _full((2, K, CH)), _full((CH,)), _full((CH * H, H)),
                  _full((H,))],
        out_specs=pl.BlockSpec((_BB, H), lambda i: (i, 0)),
        out_shape=jax.ShapeDtypeStruct((B, H), jnp.float32),
    )(e1, e2, convw_r, convb, fcw_t, fcb)


def _tc_logits(t, table, n_pad, vb):
    def body(t_ref, tab_ref, o_ref):
        dn = (((1,), (1,)), ((), ()))
        o_ref[...] = lax.dot_general(t_ref[...], tab_ref[...], dn,
                                     preferred_element_type=jnp.float32)

    return pl.pallas_call(
        body,
        grid=(B // _BB, n_pad // vb),
        in_specs=[pl.BlockSpec((_BB, H), lambda i, j: (i, 0)),
                  pl.BlockSpec((vb, H), lambda i, j: (j, 0))],
        out_specs=pl.BlockSpec((_BB, vb), lambda i, j: (i, j)),
        out_shape=jax.ShapeDtypeStruct((B, n_pad), jnp.float32),
    )(t, table)


@jax.jit
def kernel(ent_embeds, rel_embeds, W_r1, W_sl1, W_el1, W_r2, W_sl2, W_el2,
           lin_w, lin_b, gru_w_ih, gru_w_hh, gru_b_ih, gru_b_hh, convR_w,
           convR_b, fcR_w, fcR_b, convE_w, convE_b, fcE_w, fcE_b, edge_index,
           edge_rel, subj, rel, obj):
    npad = E_PAD - E
    src_r = jnp.concatenate(
        [edge_index[0].astype(jnp.int32), jnp.zeros((npad,), jnp.int32)]
    ).reshape(NW, NCHUNK, CHUNK)
    dst_r = jnp.concatenate(
        [edge_index[1].astype(jnp.int32),
         jnp.full((npad,), N_ENT, jnp.int32)]
    ).reshape(NW, NCHUNK, CHUNK)
    rel_r = jnp.concatenate(
        [edge_rel.astype(jnp.int32), jnp.full((npad,), N_REL, jnp.int32)]
    ).reshape(NW, NCHUNK, CHUNK)
    subj_i = subj.astype(jnp.int32)
    obj_i = obj.astype(jnp.int32)
    rel_i = rel.astype(jnp.int32)

    zsum = jnp.zeros((NPR, H), jnp.float32)
    zcnt = jnp.zeros((NPR, H), jnp.float32)
    zdeg = jnp.zeros((NPE, H), jnp.float32)
    zagg = jnp.zeros((NPE, H), jnp.float32)
    ones16 = jnp.ones((CHUNK, H), jnp.float32)

    ent_e = _tc_l2norm(ent_embeds)

    sums_pp, cnt_pp, deg_pp = _edge_stats(src_r, dst_r, rel_r, ent_e,
                                          zsum, zcnt, zdeg, ones16)
    sums_p = sums_pp[:, :N_REL]
    cnt_p = cnt_pp[:, :N_REL]
    deg0 = deg_pp[0, :N_ENT]
    deg1 = deg_pp[1, :N_ENT]

    n_rel, relW1, relW2 = _tc_relgru(rel_embeds, sums_p, cnt_p, gru_w_ih,
                                     gru_w_hh, gru_b_ih, gru_b_hh, W_r1, W_r2)
    zrel = jnp.zeros((NPR - N_REL, H), jnp.float32)
    relW1 = jnp.concatenate([relW1, zrel])
    relW2 = jnp.concatenate([relW2, zrel])

    hW1, self1, u = _tc_entprep(ent_e, deg0, deg1, W_r1, W_sl1, W_el1,
                                lin_w, lin_b)

    agg1_p = _edge_agg(src_r, dst_r, rel_r, hW1, relW1, zagg)
    hW2, self2 = _tc_layer2prep(agg1_p[0, :N_ENT], agg1_p[1, :N_ENT], self1,
                                deg0, deg1, W_r2, W_sl2, W_el2)

    agg2_p = _edge_agg(src_r, dst_r, rel_r, hW2, relW2, zagg)
    ent_f = _tc_entfinal(agg2_p[0, :N_ENT], agg2_p[1, :N_ENT], self2, deg0,
                         deg1, ent_e, u)

    e1, e2, rg = _gather3(ent_f, n_rel, subj_i, obj_i, rel_i)

    convR_r = convR_w.transpose(1, 2, 0)
    convE_r = convE_w.transpose(1, 2, 0)

    tR = _tc_dec_feat(e1, e2, convR_r, convR_b, fcR_w.T, fcR_b)
    tE = _tc_dec_feat(e1, rg, convE_r, convE_b, fcE_w.T, fcE_b)

    nrel_pad = jnp.concatenate(
        [n_rel, jnp.zeros((NPR - N_REL, H), jnp.float32)])
    entf_pad = jnp.concatenate(
        [ent_f, jnp.zeros((NPE - N_ENT, H), jnp.float32)])
    rel_logit = _tc_logits(tR, nrel_pad, NPR, NPR)[:, :N_REL]
    obj_logit = _tc_logits(tE, entf_pad, NPE, 1280)[:, :N_ENT]
    return (obj_logit, rel_logit)
